# Initial kernel scaffold; baseline (speedup 1.0000x reference)
import functools

import jax
import jax.numpy as jnp
from jax import lax
from jax.experimental import pallas as pl
from jax.experimental.pallas import tpu as pltpu
from jax.experimental.pallas import tpu_sc as plsc

_N = 10000
_D = 128
_E = 320000
_B = 16384
_NC = 2
_NS = 16
_NW = _NC * _NS
_EPW = _E // _NW
_CH = 80
_NCH = _EPW // _CH
_RPT = _N // _NS
_ZB = 25


def _seg_sum_sc(feats, edges):
    mesh = plsc.VectorSubcoreMesh(core_axis_name="c", subcore_axis_name="s")

    @functools.partial(
        pl.kernel,
        out_type=(jax.ShapeDtypeStruct((_NC, _N, _D), jnp.float32),
                  jax.ShapeDtypeStruct((_NC, _N), jnp.float32)),
        mesh=mesh,
        scratch_types=[
            pltpu.VMEM((_CH,), jnp.int32),
            pltpu.VMEM((_CH,), jnp.int32),
            pltpu.VMEM((_CH, _D), jnp.float32),
            pltpu.VMEM((_CH,), jnp.float32),
            pltpu.VMEM((_ZB, _D), jnp.float32),
            pltpu.VMEM((400,), jnp.float32),
            pltpu.VMEM_SHARED((_N, _D), jnp.float32),
            pltpu.VMEM_SHARED((_N,), jnp.float32),
        ],
    )
    def k(feats_hbm, edges_hbm, acc_hbm, deg_hbm, src_v, dst_v, rows_v,
          ones_v, zrow_v, zdeg_v, acc_sh, deg_sh):
        cid = lax.axis_index("c")
        sid = lax.axis_index("s")
        wid = cid * _NS + sid

        z16 = jnp.zeros((16,), jnp.float32)
        o16 = jnp.ones((16,), jnp.float32)

        @pl.loop(0, _ZB)
        def _(r):
            @pl.loop(0, _D, step=16)
            def _(c):
                zrow_v[r, pl.ds(c, 16)] = z16

        @pl.loop(0, 400, step=16)
        def _(i):
            zdeg_v[pl.ds(i, 16)] = z16

        @pl.loop(0, _CH, step=16)
        def _(i):
            ones_v[pl.ds(i, 16)] = o16

        @pl.loop(0, _RPT // _ZB)
        def _(i):
            pltpu.sync_copy(zrow_v,
                            acc_sh.at[pl.ds(sid * _RPT + i * _ZB, _ZB)])

        @pl.when(sid == 0)
        def _():
            @pl.loop(0, _N // 400)
            def _(i):
                pltpu.sync_copy(zdeg_v, deg_sh.at[pl.ds(i * 400, 400)])

        plsc.subcore_barrier()

        @pl.loop(0, _NCH)
        def _(i):
            base = wid * _EPW + i * _CH
            pltpu.sync_copy(edges_hbm.at[0, pl.ds(base, _CH)], src_v)
            pltpu.sync_copy(edges_hbm.at[1, pl.ds(base, _CH)], dst_v)
            pltpu.sync_copy(feats_hbm.at[src_v], rows_v)
            pltpu.sync_copy(rows_v, acc_sh.at[dst_v], add=True)
            pltpu.sync_copy(ones_v, deg_sh.at[dst_v], add=True)

        plsc.subcore_barrier()

        pltpu.sync_copy(acc_sh.at[pl.ds(sid * _RPT, _RPT)],
                        acc_hbm.at[cid, pl.ds(sid * _RPT, _RPT)])

        @pl.when(sid == 0)
        def _():
            pltpu.sync_copy(deg_sh, deg_hbm.at[cid])

    return k(feats, edges)


def _gather2_sc(tab_u, idx_u, tab_s, idx_s):
    mesh = plsc.VectorSubcoreMesh(core_axis_name="c", subcore_axis_name="s")
    ipw = _B // _NW
    gch = 128
    ng = ipw // gch

    @functools.partial(
        pl.kernel,
        out_type=(jax.ShapeDtypeStruct((_B, _D), jnp.float32),
                  jax.ShapeDtypeStruct((_B, _D), jnp.float32)),
        mesh=mesh,
        scratch_types=[
            pltpu.VMEM((gch,), jnp.int32),
            pltpu.VMEM((gch, _D), jnp.float32),
        ],
    )
    def k(tu_hbm, iu_hbm, ts_hbm, is_hbm, ou_hbm, os_hbm, idx_v, rows_v):
        cid = lax.axis_index("c")
        sid = lax.axis_index("s")
        wid = cid * _NS + sid
        for t, ix, o in ((tu_hbm, iu_hbm, ou_hbm), (ts_hbm, is_hbm, os_hbm)):
            @functools.partial(pl.loop, 0, ng)
            def _(i, t=t, ix=ix, o=o):
                base = wid * ipw + i * gch
                pltpu.sync_copy(ix.at[pl.ds(base, gch)], idx_v)
                pltpu.sync_copy(t.at[idx_v], rows_v)
                pltpu.sync_copy(rows_v, o.at[pl.ds(base, gch)])

    return k(tab_u, idx_u, tab_s, idx_s)


_LN_EPS = 1e-5


def _dense_body(acc_ref, deg_ref, feats_ref, w_ref, b_ref, g_ref, beta_ref,
                out_ref):
    agg = acc_ref[0] + acc_ref[1] + feats_ref[...]
    deg = deg_ref[0] + deg_ref[1] + 1.0
    h = agg / deg
    h = jnp.dot(h, w_ref[...], preferred_element_type=jnp.float32) + b_ref[...]
    mu = jnp.mean(h, axis=-1, keepdims=True)
    var = jnp.mean((h - mu) ** 2, axis=-1, keepdims=True)
    h = (h - mu) * lax.rsqrt(var + _LN_EPS) * g_ref[...] + beta_ref[...]
    out_ref[...] = jnp.where(h > 0, h, jnp.expm1(h))


def _dense_tc(acc, deg, feats, w, b, g, beta, interpret=False):
    r = 400
    return pl.pallas_call(
        _dense_body,
        grid=(_N // r,),
        in_specs=[
            pl.BlockSpec((_NC, r, _D), lambda i: (0, i, 0)),
            pl.BlockSpec((_NC, r, 1), lambda i: (0, i, 0)),
            pl.BlockSpec((r, _D), lambda i: (i, 0)),
            pl.BlockSpec((_D, _D), lambda i: (0, 0)),
            pl.BlockSpec((1, _D), lambda i: (0, 0)),
            pl.BlockSpec((1, _D), lambda i: (0, 0)),
            pl.BlockSpec((1, _D), lambda i: (0, 0)),
        ],
        out_specs=pl.BlockSpec((r, _D), lambda i: (i, 0)),
        out_shape=jax.ShapeDtypeStruct((_N, _D), jnp.float32),
        interpret=interpret,
    )(acc, deg[..., None], feats, w, b.reshape(1, _D), g.reshape(1, _D),
      beta.reshape(1, _D))


def _mlp_body(u_ref, s_ref, w1u_ref, w1s_ref, b1_ref, g1_ref, beta1_ref,
              w2_ref, b2_ref, g2_ref, beta2_ref, w3_ref, b3_ref, out_ref):
    h = (jnp.dot(u_ref[...], w1u_ref[...], preferred_element_type=jnp.float32)
         + jnp.dot(s_ref[...], w1s_ref[...], preferred_element_type=jnp.float32)
         + b1_ref[...])
    mu = jnp.mean(h, axis=-1, keepdims=True)
    var = jnp.mean((h - mu) ** 2, axis=-1, keepdims=True)
    h = (h - mu) * lax.rsqrt(var + _LN_EPS) * g1_ref[...] + beta1_ref[...]
    h = jnp.maximum(h, 0.0)
    h = jnp.dot(h, w2_ref[...], preferred_element_type=jnp.float32) + b2_ref[...]
    mu = jnp.mean(h, axis=-1, keepdims=True)
    var = jnp.mean((h - mu) ** 2, axis=-1, keepdims=True)
    h = (h - mu) * lax.rsqrt(var + _LN_EPS) * g2_ref[...] + beta2_ref[...]
    h = jnp.maximum(h, 0.0)
    z = jnp.dot(h, w3_ref[...], preferred_element_type=jnp.float32) + b3_ref[...]
    out_ref[...] = jax.nn.sigmoid(z)


def _mlp_tc(u, s, w1, b1, g1, beta1, w2, b2, g2, beta2, w3, b3,
            interpret=False):
    r = 2048
    hid = w2.shape[0]
    return pl.pallas_call(
        _mlp_body,
        grid=(_B // r,),
        in_specs=[
            pl.BlockSpec((r, _D), lambda i: (i, 0)),
            pl.BlockSpec((r, _D), lambda i: (i, 0)),
            pl.BlockSpec((_D, hid), lambda i: (0, 0)),
            pl.BlockSpec((_D, hid), lambda i: (0, 0)),
            pl.BlockSpec((1, hid), lambda i: (0, 0)),
            pl.BlockSpec((1, hid), lambda i: (0, 0)),
            pl.BlockSpec((1, hid), lambda i: (0, 0)),
            pl.BlockSpec((hid, hid), lambda i: (0, 0)),
            pl.BlockSpec((1, hid), lambda i: (0, 0)),
            pl.BlockSpec((1, hid), lambda i: (0, 0)),
            pl.BlockSpec((1, hid), lambda i: (0, 0)),
            pl.BlockSpec((hid, 1), lambda i: (0, 0)),
            pl.BlockSpec((1, 1), lambda i: (0, 0)),
        ],
        out_specs=pl.BlockSpec((r, 1), lambda i: (i, 0)),
        out_shape=jax.ShapeDtypeStruct((_B, 1), jnp.float32),
        interpret=interpret,
    )(u, s, w1[:_D], w1[_D:], b1.reshape(1, -1), g1.reshape(1, -1),
      beta1.reshape(1, -1), w2, b2.reshape(1, -1), g2.reshape(1, -1),
      beta2.reshape(1, -1), w3, b3.reshape(1, 1))


def kernel(params, user_edges, serv_edges, userIdx, itemIdx):
    p = params
    side_out = {}
    for side, emb, edges in (("user", p["user_emb"], user_edges),
                             ("serv", p["serv_emb"], serv_edges)):
        feats = emb
        deg = None
        for l in range(2):
            acc, d = _seg_sum_sc(feats, edges)
            if deg is None:
                deg = d
            feats = _dense_tc(acc, deg, feats, p[f"{side}_W{l}"],
                              p[f"{side}_b{l}"], p[f"{side}_g{l}"],
                              p[f"{side}_beta{l}"])
        side_out[side] = feats
    u, s = _gather2_sc(side_out["user"], userIdx, side_out["serv"], itemIdx)
    est = _mlp_tc(u, s, p["W1"], p["b1"], p["g1"], p["beta1"], p["W2"],
                  p["b2"], p["g2"], p["beta2"], p["W3"], p["b3"])
    return est.reshape(_B)


# speedup vs baseline: 3.5216x; 3.5216x over previous
import functools

import jax
import jax.numpy as jnp
from jax import lax
from jax.experimental import pallas as pl
from jax.experimental.pallas import tpu as pltpu
from jax.experimental.pallas import tpu_sc as plsc

_N = 10000
_D = 128
_E = 320000
_B = 16384
_NC = 2
_NS = 16
_NW = _NC * _NS
_EPW = _E // _NW
_CH = 80
_NCH = _EPW // _CH
_NP = 10240
_RPT = _NP // _NS
_ZB = 40


def _seg_sum_sc(feats, src, dst):
    mesh = plsc.VectorSubcoreMesh(core_axis_name="c", subcore_axis_name="s")

    @functools.partial(
        pl.kernel,
        out_type=(jax.ShapeDtypeStruct((_NC, _NP, _D), jnp.float32),
                  jax.ShapeDtypeStruct((_NC * _NP,), jnp.float32)),
        mesh=mesh,
        scratch_types=[
            pltpu.VMEM((_CH,), jnp.int32),
            pltpu.VMEM((_CH,), jnp.int32),
            pltpu.VMEM((_CH, _D), jnp.float32),
            pltpu.VMEM((_CH,), jnp.float32),
            pltpu.VMEM((_ZB, _D), jnp.float32),
            pltpu.VMEM((_RPT,), jnp.float32),
            pltpu.VMEM_SHARED((_NP, _D), jnp.float32),
            pltpu.VMEM_SHARED((_NP,), jnp.float32),
        ],
    )
    def k(feats_hbm, src_hbm, dst_hbm, acc_hbm, deg_hbm, src_v, dst_v, rows_v,
          ones_v, zrow_v, zdeg_v, acc_sh, deg_sh):
        cid = lax.axis_index("c")
        sid = lax.axis_index("s")
        wid = cid * _NS + sid

        z16 = jnp.zeros((16,), jnp.float32)
        o16 = jnp.ones((16,), jnp.float32)

        @pl.loop(0, _ZB)
        def _(r):
            @pl.loop(0, _D, step=16)
            def _(c):
                zrow_v[r, pl.ds(c, 16)] = z16

        @pl.loop(0, _RPT, step=16)
        def _(i):
            zdeg_v[pl.ds(i, 16)] = z16

        @pl.loop(0, _CH, step=16)
        def _(i):
            ones_v[pl.ds(i, 16)] = o16

        @pl.loop(0, _RPT // _ZB)
        def _(i):
            pltpu.sync_copy(zrow_v,
                            acc_sh.at[pl.ds(sid * _RPT + i * _ZB, _ZB)])

        pltpu.sync_copy(zdeg_v, deg_sh.at[pl.ds(sid * _RPT, _RPT)])

        plsc.subcore_barrier()

        @pl.loop(0, _NCH)
        def _(i):
            base = wid * _EPW + i * _CH
            pltpu.sync_copy(src_hbm.at[pl.ds(base, _CH)], src_v)
            pltpu.sync_copy(dst_hbm.at[pl.ds(base, _CH)], dst_v)
            pltpu.sync_copy(feats_hbm.at[src_v], rows_v)
            pltpu.sync_copy(rows_v, acc_sh.at[dst_v], add=True)
            pltpu.sync_copy(ones_v, deg_sh.at[dst_v], add=True)

        plsc.subcore_barrier()

        pltpu.sync_copy(acc_sh.at[pl.ds(sid * _RPT, _RPT)],
                        acc_hbm.at[cid, pl.ds(sid * _RPT, _RPT)])
        pltpu.sync_copy(deg_sh.at[pl.ds(sid * _RPT, _RPT)],
                        deg_hbm.at[pl.ds(cid * _NP + sid * _RPT, _RPT)])

    return k(feats, src, dst)


def _gather2_sc(tab_u, idx_u, tab_s, idx_s):
    mesh = plsc.VectorSubcoreMesh(core_axis_name="c", subcore_axis_name="s")
    ipw = _B // _NW
    gch = 128
    ng = ipw // gch

    @functools.partial(
        pl.kernel,
        out_type=(jax.ShapeDtypeStruct((_B, _D), jnp.float32),
                  jax.ShapeDtypeStruct((_B, _D), jnp.float32)),
        mesh=mesh,
        scratch_types=[
            pltpu.VMEM((gch,), jnp.int32),
            pltpu.VMEM((gch, _D), jnp.float32),
        ],
    )
    def k(tu_hbm, iu_hbm, ts_hbm, is_hbm, ou_hbm, os_hbm, idx_v, rows_v):
        cid = lax.axis_index("c")
        sid = lax.axis_index("s")
        wid = cid * _NS + sid
        for t, ix, o in ((tu_hbm, iu_hbm, ou_hbm), (ts_hbm, is_hbm, os_hbm)):
            def body(i, t=t, ix=ix, o=o):
                base = wid * ipw + i * gch
                pltpu.sync_copy(ix.at[pl.ds(base, gch)], idx_v)
                pltpu.sync_copy(t.at[idx_v], rows_v)
                pltpu.sync_copy(rows_v, o.at[pl.ds(base, gch)])
            pl.loop(0, ng)(body)

    return k(tab_u, idx_u, tab_s, idx_s)


_LN_EPS = 1e-5


def _dense_body(acc_ref, deg_ref, feats_ref, w_ref, b_ref, g_ref, beta_ref,
                out_ref):
    agg = acc_ref[0] + acc_ref[1] + feats_ref[...]
    deg = deg_ref[0] + deg_ref[1] + 1.0
    h = agg / deg
    h = jnp.dot(h, w_ref[...], preferred_element_type=jnp.float32) + b_ref[...]
    mu = jnp.mean(h, axis=-1, keepdims=True)
    var = jnp.mean((h - mu) ** 2, axis=-1, keepdims=True)
    h = (h - mu) * lax.rsqrt(var + _LN_EPS) * g_ref[...] + beta_ref[...]
    out_ref[...] = jnp.where(h > 0, h, jnp.exp(jnp.minimum(h, 0.0)) - 1.0)


def _dense_tc(acc, deg, feats, w, b, g, beta, interpret=False):
    r = 400
    return pl.pallas_call(
        _dense_body,
        grid=(_N // r,),
        in_specs=[
            pl.BlockSpec((_NC, r, _D), lambda i: (0, i, 0)),
            pl.BlockSpec((_NC, r, 1), lambda i: (0, i, 0)),
            pl.BlockSpec((r, _D), lambda i: (i, 0)),
            pl.BlockSpec((_D, _D), lambda i: (0, 0)),
            pl.BlockSpec((1, _D), lambda i: (0, 0)),
            pl.BlockSpec((1, _D), lambda i: (0, 0)),
            pl.BlockSpec((1, _D), lambda i: (0, 0)),
        ],
        out_specs=pl.BlockSpec((r, _D), lambda i: (i, 0)),
        out_shape=jax.ShapeDtypeStruct((_N, _D), jnp.float32),
        interpret=interpret,
    )(acc, deg[..., None], feats, w, b.reshape(1, _D), g.reshape(1, _D),
      beta.reshape(1, _D))


def _mlp_body(u_ref, s_ref, w1u_ref, w1s_ref, b1_ref, g1_ref, beta1_ref,
              w2_ref, b2_ref, g2_ref, beta2_ref, w3_ref, b3_ref, out_ref):
    h = (jnp.dot(u_ref[...], w1u_ref[...], preferred_element_type=jnp.float32)
         + jnp.dot(s_ref[...], w1s_ref[...], preferred_element_type=jnp.float32)
         + b1_ref[...])
    mu = jnp.mean(h, axis=-1, keepdims=True)
    var = jnp.mean((h - mu) ** 2, axis=-1, keepdims=True)
    h = (h - mu) * lax.rsqrt(var + _LN_EPS) * g1_ref[...] + beta1_ref[...]
    h = jnp.maximum(h, 0.0)
    h = jnp.dot(h, w2_ref[...], preferred_element_type=jnp.float32) + b2_ref[...]
    mu = jnp.mean(h, axis=-1, keepdims=True)
    var = jnp.mean((h - mu) ** 2, axis=-1, keepdims=True)
    h = (h - mu) * lax.rsqrt(var + _LN_EPS) * g2_ref[...] + beta2_ref[...]
    h = jnp.maximum(h, 0.0)
    z = jnp.dot(h, w3_ref[...], preferred_element_type=jnp.float32) + b3_ref[...]
    out_ref[...] = jax.nn.sigmoid(z)


def _mlp_tc(u, s, w1, b1, g1, beta1, w2, b2, g2, beta2, w3, b3,
            interpret=False):
    r = 2048
    hid = w2.shape[0]
    return pl.pallas_call(
        _mlp_body,
        grid=(_B // r,),
        in_specs=[
            pl.BlockSpec((r, _D), lambda i: (i, 0)),
            pl.BlockSpec((r, _D), lambda i: (i, 0)),
            pl.BlockSpec((_D, hid), lambda i: (0, 0)),
            pl.BlockSpec((_D, hid), lambda i: (0, 0)),
            pl.BlockSpec((1, hid), lambda i: (0, 0)),
            pl.BlockSpec((1, hid), lambda i: (0, 0)),
            pl.BlockSpec((1, hid), lambda i: (0, 0)),
            pl.BlockSpec((hid, hid), lambda i: (0, 0)),
            pl.BlockSpec((1, hid), lambda i: (0, 0)),
            pl.BlockSpec((1, hid), lambda i: (0, 0)),
            pl.BlockSpec((1, hid), lambda i: (0, 0)),
            pl.BlockSpec((hid, 1), lambda i: (0, 0)),
            pl.BlockSpec((1, 1), lambda i: (0, 0)),
        ],
        out_specs=pl.BlockSpec((r, 1), lambda i: (i, 0)),
        out_shape=jax.ShapeDtypeStruct((_B, 1), jnp.float32),
        interpret=interpret,
    )(u, s, w1[:_D], w1[_D:], b1.reshape(1, -1), g1.reshape(1, -1),
      beta1.reshape(1, -1), w2, b2.reshape(1, -1), g2.reshape(1, -1),
      beta2.reshape(1, -1), w3, b3.reshape(1, 1))


def kernel(params, user_edges, serv_edges, userIdx, itemIdx):
    p = params
    side_out = {}
    for side, emb, edges in (("user", p["user_emb"], user_edges),
                             ("serv", p["serv_emb"], serv_edges)):
        src, dst = edges[0], edges[1]
        feats = emb
        deg = None
        for l in range(2):
            acc, d = _seg_sum_sc(feats, src, dst)
            if deg is None:
                deg = d.reshape(_NC, _NP)
            feats = _dense_tc(acc, deg, feats, p[f"{side}_W{l}"],
                              p[f"{side}_b{l}"], p[f"{side}_g{l}"],
                              p[f"{side}_beta{l}"])
        side_out[side] = feats
    u, s = _gather2_sc(side_out["user"], userIdx, side_out["serv"], itemIdx)
    est = _mlp_tc(u, s, p["W1"], p["b1"], p["g1"], p["beta1"], p["W2"],
                  p["b2"], p["g2"], p["beta2"], p["W3"], p["b3"])
    return est.reshape(_B)


# speedup vs baseline: 7.3444x; 2.0855x over previous
import functools

import jax
import jax.numpy as jnp
from jax import lax
from jax.experimental import pallas as pl
from jax.experimental.pallas import tpu as pltpu
from jax.experimental.pallas import tpu_sc as plsc

_N = 10000
_D = 128
_E = 320000
_B = 16384
_NC = 2
_NS = 16
_NW = _NC * _NS
_CH = 125
_NCH = 80
_NROW = _E // _CH
_NP = 10240
_RPT = _NP // _NS
_ZB = 8
_NB = 2
_IB = 40


def _seg_sum_sc(feats, src2d, dst2d, with_deg):
    mesh = plsc.VectorSubcoreMesh(core_axis_name="c", subcore_axis_name="s")

    out_type = [jax.ShapeDtypeStruct((_NC, _NP, _D), jnp.float32)]
    if with_deg:
        out_type.append(jax.ShapeDtypeStruct((_NC * _NP,), jnp.float32))

    scratch = [
        pltpu.VMEM((_IB, _CH), jnp.int32),
        pltpu.VMEM((_IB, _CH), jnp.int32),
        pltpu.VMEM((_CH, _D), jnp.float32),
        pltpu.VMEM((_CH, _D), jnp.float32),
        pltpu.VMEM((128,), jnp.float32),
        pltpu.VMEM((_ZB, _D), jnp.float32),
        pltpu.VMEM((_RPT,), jnp.float32),
        pltpu.VMEM_SHARED((_NP, _D), jnp.float32),
        pltpu.VMEM_SHARED((_NP,), jnp.float32),
        pltpu.SemaphoreType.DMA,
        pltpu.SemaphoreType.DMA,
        pltpu.SemaphoreType.DMA,
        pltpu.SemaphoreType.DMA,
        pltpu.SemaphoreType.DMA,
        pltpu.SemaphoreType.DMA,
    ]

    @functools.partial(pl.kernel, out_type=tuple(out_type), mesh=mesh,
                       scratch_types=scratch)
    def k(feats_hbm, src_hbm, dst_hbm, acc_hbm, *rest):
        if with_deg:
            deg_hbm = rest[0]
            rest = rest[1:]
        (srcs_v, dsts_v, rows_a, rows_b, ones_v, zrow_v, zdeg_v,
         acc_sh, deg_sh, sem_m, sem_ga, sem_gb, sem_sa, sem_sb,
         sem_d) = rest
        cid = lax.axis_index("c")
        sid = lax.axis_index("s")
        wid = cid * _NS + sid
        bufs = ((rows_a, sem_ga, sem_sa), (rows_b, sem_gb, sem_sb))

        z16 = jnp.zeros((16,), jnp.float32)

        @pl.loop(0, _ZB)
        def _(r):
            @pl.loop(0, _D, step=16)
            def _(c):
                zrow_v[r, pl.ds(c, 16)] = z16

        if with_deg:
            o16 = jnp.ones((16,), jnp.float32)

            @pl.loop(0, _RPT, step=16)
            def _(i):
                zdeg_v[pl.ds(i, 16)] = z16

            @pl.loop(0, 128, step=16)
            def _(i):
                ones_v[pl.ds(i, 16)] = o16

        for i in range(_RPT // _ZB):
            pltpu.async_copy(
                zrow_v, acc_sh.at[pl.ds(sid * _RPT + i * _ZB, _ZB)], sem_m)
        if with_deg:
            pltpu.async_copy(zdeg_v, deg_sh.at[pl.ds(sid * _RPT, _RPT)],
                             sem_m)
        for i in range(_RPT // _ZB):
            pltpu.make_async_copy(
                zrow_v, acc_sh.at[pl.ds(sid * _RPT + i * _ZB, _ZB)],
                sem_m).wait()
        if with_deg:
            pltpu.make_async_copy(zdeg_v, deg_sh.at[pl.ds(sid * _RPT, _RPT)],
                                  sem_m).wait()

        plsc.subcore_barrier()

        @pl.loop(0, _NCH, step=_IB)
        def _(t0):
            row0 = wid * _NCH + t0
            pltpu.sync_copy(src_hbm.at[pl.ds(row0, _IB)], srcs_v)
            pltpu.sync_copy(dst_hbm.at[pl.ds(row0, _IB)], dsts_v)

            for b, (buf, sem_g, _) in enumerate(bufs):
                pltpu.async_copy(feats_hbm.at[srcs_v.at[b]], buf, sem_g)

            @pl.loop(0, _IB, step=_NB)
            def _(t):
                for b, (buf, sem_g, sem_s) in enumerate(bufs):
                    j = t + b
                    pltpu.make_async_copy(feats_hbm.at[srcs_v.at[j]], buf,
                                          sem_g).wait()
                    pltpu.async_copy(buf, acc_sh.at[dsts_v.at[j]], sem_s,
                                     add=True)
                    if with_deg:
                        pltpu.async_copy(ones_v.at[pl.ds(0, _CH)],
                                         deg_sh.at[dsts_v.at[j]], sem_d,
                                         add=True)
                for b, (buf, sem_g, sem_s) in enumerate(bufs):
                    j = t + b
                    pltpu.make_async_copy(buf, acc_sh.at[dsts_v.at[j]],
                                          sem_s).wait()

                    @pl.when(t + _NB < _IB)
                    def _(j=j, buf=buf, sem_g=sem_g):
                        pltpu.async_copy(feats_hbm.at[srcs_v.at[j + _NB]],
                                         buf, sem_g)

            if with_deg:
                @pl.loop(0, _IB)
                def _(j):
                    pltpu.make_async_copy(ones_v.at[pl.ds(0, _CH)],
                                          deg_sh.at[dsts_v.at[j]],
                                          sem_d).wait()

        plsc.subcore_barrier()

        pltpu.sync_copy(acc_sh.at[pl.ds(sid * _RPT, _RPT)],
                        acc_hbm.at[cid, pl.ds(sid * _RPT, _RPT)])
        if with_deg:
            pltpu.sync_copy(deg_sh.at[pl.ds(sid * _RPT, _RPT)],
                            deg_hbm.at[pl.ds(cid * _NP + sid * _RPT, _RPT)])

    return k(feats, src2d, dst2d)


def _gather2_sc(tab_u, idx_u, tab_s, idx_s):
    mesh = plsc.VectorSubcoreMesh(core_axis_name="c", subcore_axis_name="s")
    ipw = _B // _NW
    gch = 128
    ng = ipw // gch

    @functools.partial(
        pl.kernel,
        out_type=(jax.ShapeDtypeStruct((_B, _D), jnp.float32),
                  jax.ShapeDtypeStruct((_B, _D), jnp.float32)),
        mesh=mesh,
        scratch_types=[
            pltpu.VMEM((gch,), jnp.int32),
            pltpu.VMEM((gch, _D), jnp.float32),
        ],
    )
    def k(tu_hbm, iu_hbm, ts_hbm, is_hbm, ou_hbm, os_hbm, idx_v, rows_v):
        cid = lax.axis_index("c")
        sid = lax.axis_index("s")
        wid = cid * _NS + sid
        for t, ix, o in ((tu_hbm, iu_hbm, ou_hbm), (ts_hbm, is_hbm, os_hbm)):
            def body(i, t=t, ix=ix, o=o):
                base = wid * ipw + i * gch
                pltpu.sync_copy(ix.at[pl.ds(base, gch)], idx_v)
                pltpu.sync_copy(t.at[idx_v], rows_v)
                pltpu.sync_copy(rows_v, o.at[pl.ds(base, gch)])
            pl.loop(0, ng)(body)

    return k(tab_u, idx_u, tab_s, idx_s)


_LN_EPS = 1e-5


def _dense_body(acc_ref, deg_ref, feats_ref, w_ref, b_ref, g_ref, beta_ref,
                out_ref):
    agg = acc_ref[0] + acc_ref[1] + feats_ref[...]
    deg = deg_ref[0] + deg_ref[1] + 1.0
    h = agg / deg
    h = jnp.dot(h, w_ref[...], preferred_element_type=jnp.float32) + b_ref[...]
    mu = jnp.mean(h, axis=-1, keepdims=True)
    var = jnp.mean((h - mu) ** 2, axis=-1, keepdims=True)
    h = (h - mu) * lax.rsqrt(var + _LN_EPS) * g_ref[...] + beta_ref[...]
    out_ref[...] = jnp.where(h > 0, h, jnp.exp(jnp.minimum(h, 0.0)) - 1.0)


def _dense_tc(acc, deg, feats, w, b, g, beta, interpret=False):
    r = 400
    return pl.pallas_call(
        _dense_body,
        grid=(_N // r,),
        in_specs=[
            pl.BlockSpec((_NC, r, _D), lambda i: (0, i, 0)),
            pl.BlockSpec((_NC, r, 1), lambda i: (0, i, 0)),
            pl.BlockSpec((r, _D), lambda i: (i, 0)),
            pl.BlockSpec((_D, _D), lambda i: (0, 0)),
            pl.BlockSpec((1, _D), lambda i: (0, 0)),
            pl.BlockSpec((1, _D), lambda i: (0, 0)),
            pl.BlockSpec((1, _D), lambda i: (0, 0)),
        ],
        out_specs=pl.BlockSpec((r, _D), lambda i: (i, 0)),
        out_shape=jax.ShapeDtypeStruct((_N, _D), jnp.float32),
        interpret=interpret,
    )(acc, deg[..., None], feats, w, b.reshape(1, _D), g.reshape(1, _D),
      beta.reshape(1, _D))


def _mlp_body(u_ref, s_ref, w1u_ref, w1s_ref, b1_ref, g1_ref, beta1_ref,
              w2_ref, b2_ref, g2_ref, beta2_ref, w3_ref, b3_ref, out_ref):
    h = (jnp.dot(u_ref[...], w1u_ref[...], preferred_element_type=jnp.float32)
         + jnp.dot(s_ref[...], w1s_ref[...], preferred_element_type=jnp.float32)
         + b1_ref[...])
    mu = jnp.mean(h, axis=-1, keepdims=True)
    var = jnp.mean((h - mu) ** 2, axis=-1, keepdims=True)
    h = (h - mu) * lax.rsqrt(var + _LN_EPS) * g1_ref[...] + beta1_ref[...]
    h = jnp.maximum(h, 0.0)
    h = jnp.dot(h, w2_ref[...], preferred_element_type=jnp.float32) + b2_ref[...]
    mu = jnp.mean(h, axis=-1, keepdims=True)
    var = jnp.mean((h - mu) ** 2, axis=-1, keepdims=True)
    h = (h - mu) * lax.rsqrt(var + _LN_EPS) * g2_ref[...] + beta2_ref[...]
    h = jnp.maximum(h, 0.0)
    z = jnp.dot(h, w3_ref[...], preferred_element_type=jnp.float32) + b3_ref[...]
    out_ref[...] = jax.nn.sigmoid(z)


def _mlp_tc(u, s, w1, b1, g1, beta1, w2, b2, g2, beta2, w3, b3,
            interpret=False):
    r = 2048
    hid = w2.shape[0]
    return pl.pallas_call(
        _mlp_body,
        grid=(_B // r,),
        in_specs=[
            pl.BlockSpec((r, _D), lambda i: (i, 0)),
            pl.BlockSpec((r, _D), lambda i: (i, 0)),
            pl.BlockSpec((_D, hid), lambda i: (0, 0)),
            pl.BlockSpec((_D, hid), lambda i: (0, 0)),
            pl.BlockSpec((1, hid), lambda i: (0, 0)),
            pl.BlockSpec((1, hid), lambda i: (0, 0)),
            pl.BlockSpec((1, hid), lambda i: (0, 0)),
            pl.BlockSpec((hid, hid), lambda i: (0, 0)),
            pl.BlockSpec((1, hid), lambda i: (0, 0)),
            pl.BlockSpec((1, hid), lambda i: (0, 0)),
            pl.BlockSpec((1, hid), lambda i: (0, 0)),
            pl.BlockSpec((hid, 1), lambda i: (0, 0)),
            pl.BlockSpec((1, 1), lambda i: (0, 0)),
        ],
        out_specs=pl.BlockSpec((r, 1), lambda i: (i, 0)),
        out_shape=jax.ShapeDtypeStruct((_B, 1), jnp.float32),
        interpret=interpret,
    )(u, s, w1[:_D], w1[_D:], b1.reshape(1, -1), g1.reshape(1, -1),
      beta1.reshape(1, -1), w2, b2.reshape(1, -1), g2.reshape(1, -1),
      beta2.reshape(1, -1), w3, b3.reshape(1, 1))


def kernel(params, user_edges, serv_edges, userIdx, itemIdx):
    p = params
    side_out = {}
    for side, emb, edges in (("user", p["user_emb"], user_edges),
                             ("serv", p["serv_emb"], serv_edges)):
        src2d = edges[0].reshape(_NROW, _CH)
        dst2d = edges[1].reshape(_NROW, _CH)
        feats = emb
        deg = None
        for l in range(2):
            if deg is None:
                acc, d = _seg_sum_sc(feats, src2d, dst2d, True)
                deg = d.reshape(_NC, _NP)
            else:
                (acc,) = _seg_sum_sc(feats, src2d, dst2d, False)
            feats = _dense_tc(acc, deg, feats, p[f"{side}_W{l}"],
                              p[f"{side}_b{l}"], p[f"{side}_g{l}"],
                              p[f"{side}_beta{l}"])
        side_out[side] = feats
    u, s = _gather2_sc(side_out["user"], userIdx, side_out["serv"], itemIdx)
    est = _mlp_tc(u, s, p["W1"], p["b1"], p["g1"], p["beta1"], p["W2"],
                  p["b2"], p["g2"], p["beta2"], p["W3"], p["b3"])
    return est.reshape(_B)


# speedup vs baseline: 7.4393x; 1.0129x over previous
import functools

import jax
import jax.numpy as jnp
from jax import lax
from jax.experimental import pallas as pl
from jax.experimental.pallas import tpu as pltpu
from jax.experimental.pallas import tpu_sc as plsc

_N = 10000
_D = 128
_E = 320000
_B = 16384
_NC = 2
_NS = 16
_NW = _NC * _NS
_CH = 125
_NCH = 80
_NROW = _E // _CH
_NP = 10240
_RPT = _NP // _NS
_ZB = 8
_NB = 2
_IB = 40


def _seg_sum_sc(feats, src2d, dst2d, with_deg):
    mesh = plsc.VectorSubcoreMesh(core_axis_name="c", subcore_axis_name="s")

    out_type = [jax.ShapeDtypeStruct((_NC, _NP, _D), jnp.float32)]
    if with_deg:
        out_type.append(jax.ShapeDtypeStruct((_NC * _NP,), jnp.float32))

    scratch = [
        pltpu.VMEM((_IB, _CH), jnp.int32),
        pltpu.VMEM((_IB, _CH), jnp.int32),
        pltpu.VMEM((_CH, _D), jnp.float32),
        pltpu.VMEM((_CH, _D), jnp.float32),
        pltpu.VMEM((128,), jnp.float32),
        pltpu.VMEM((_ZB, _D), jnp.float32),
        pltpu.VMEM((_RPT,), jnp.float32),
        pltpu.VMEM_SHARED((_NP, _D), jnp.float32),
        pltpu.VMEM_SHARED((_NP,), jnp.float32),
        pltpu.SemaphoreType.DMA,
        pltpu.SemaphoreType.DMA,
        pltpu.SemaphoreType.DMA,
        pltpu.SemaphoreType.DMA,
        pltpu.SemaphoreType.DMA,
        pltpu.SemaphoreType.DMA,
    ]

    @functools.partial(pl.kernel, out_type=tuple(out_type), mesh=mesh,
                       scratch_types=scratch)
    def k(feats_hbm, src_hbm, dst_hbm, acc_hbm, *rest):
        if with_deg:
            deg_hbm = rest[0]
            rest = rest[1:]
        (srcs_v, dsts_v, rows_a, rows_b, ones_v, zrow_v, zdeg_v,
         acc_sh, deg_sh, sem_m, sem_ga, sem_gb, sem_sa, sem_sb,
         sem_d) = rest
        cid = lax.axis_index("c")
        sid = lax.axis_index("s")
        wid = cid * _NS + sid
        bufs = ((rows_a, sem_ga, sem_sa), (rows_b, sem_gb, sem_sb))

        z16 = jnp.zeros((16,), jnp.float32)

        @pl.loop(0, _ZB)
        def _(r):
            @pl.loop(0, _D, step=16)
            def _(c):
                zrow_v[r, pl.ds(c, 16)] = z16

        if with_deg:
            o16 = jnp.ones((16,), jnp.float32)

            @pl.loop(0, _RPT, step=16)
            def _(i):
                zdeg_v[pl.ds(i, 16)] = z16

            @pl.loop(0, 128, step=16)
            def _(i):
                ones_v[pl.ds(i, 16)] = o16

        for i in range(_RPT // _ZB):
            pltpu.async_copy(
                zrow_v, acc_sh.at[pl.ds(sid * _RPT + i * _ZB, _ZB)], sem_m)
        if with_deg:
            pltpu.async_copy(zdeg_v, deg_sh.at[pl.ds(sid * _RPT, _RPT)],
                             sem_m)
        for i in range(_RPT // _ZB):
            pltpu.make_async_copy(
                zrow_v, acc_sh.at[pl.ds(sid * _RPT + i * _ZB, _ZB)],
                sem_m).wait()
        if with_deg:
            pltpu.make_async_copy(zdeg_v, deg_sh.at[pl.ds(sid * _RPT, _RPT)],
                                  sem_m).wait()

        plsc.subcore_barrier()

        @pl.loop(0, _NCH, step=_IB)
        def _(t0):
            row0 = wid * _NCH + t0
            pltpu.sync_copy(src_hbm.at[pl.ds(row0, _IB)], srcs_v)
            pltpu.sync_copy(dst_hbm.at[pl.ds(row0, _IB)], dsts_v)

            for b, (buf, sem_g, _) in enumerate(bufs):
                pltpu.async_copy(feats_hbm.at[srcs_v.at[b]], buf, sem_g)

            @pl.loop(0, _IB, step=_NB)
            def _(t):
                for b, (buf, sem_g, sem_s) in enumerate(bufs):
                    j = t + b
                    pltpu.make_async_copy(feats_hbm.at[srcs_v.at[j]], buf,
                                          sem_g).wait()
                    pltpu.async_copy(buf, acc_sh.at[dsts_v.at[j]], sem_s,
                                     add=True)
                    if with_deg:
                        pltpu.async_copy(ones_v.at[pl.ds(0, _CH)],
                                         deg_sh.at[dsts_v.at[j]], sem_d,
                                         add=True)
                for b, (buf, sem_g, sem_s) in enumerate(bufs):
                    j = t + b
                    pltpu.make_async_copy(buf, acc_sh.at[dsts_v.at[j]],
                                          sem_s).wait()

                    @pl.when(t + _NB < _IB)
                    def _(j=j, buf=buf, sem_g=sem_g):
                        pltpu.async_copy(feats_hbm.at[srcs_v.at[j + _NB]],
                                         buf, sem_g)

            if with_deg:
                @pl.loop(0, _IB)
                def _(j):
                    pltpu.make_async_copy(ones_v.at[pl.ds(0, _CH)],
                                          deg_sh.at[dsts_v.at[j]],
                                          sem_d).wait()

        plsc.subcore_barrier()

        pltpu.sync_copy(acc_sh.at[pl.ds(sid * _RPT, _RPT)],
                        acc_hbm.at[cid, pl.ds(sid * _RPT, _RPT)])
        if with_deg:
            pltpu.sync_copy(deg_sh.at[pl.ds(sid * _RPT, _RPT)],
                            deg_hbm.at[pl.ds(cid * _NP + sid * _RPT, _RPT)])

    return k(feats, src2d, dst2d)


def _gather2_sc(tab_u, idx_u, tab_s, idx_s):
    mesh = plsc.VectorSubcoreMesh(core_axis_name="c", subcore_axis_name="s")
    ipw = _B // _NW
    gch = 128
    nbuf = 6
    nch = 2 * (ipw // gch)

    scratch = ([pltpu.VMEM((ipw,), jnp.int32)] * 2 +
               [pltpu.VMEM((gch, _D), jnp.float32)] * nbuf +
               [pltpu.SemaphoreType.DMA] * (1 + 2 * nbuf))

    @functools.partial(
        pl.kernel,
        out_type=(jax.ShapeDtypeStruct((_B, _D), jnp.float32),
                  jax.ShapeDtypeStruct((_B, _D), jnp.float32)),
        mesh=mesh,
        scratch_types=scratch,
    )
    def k(tu_hbm, iu_hbm, ts_hbm, is_hbm, ou_hbm, os_hbm, *rest):
        iu_v, is_v = rest[0], rest[1]
        bufs = rest[2:2 + nbuf]
        sem_m = rest[2 + nbuf]
        sem_g = rest[3 + nbuf:3 + 2 * nbuf]
        sem_w = rest[3 + 2 * nbuf:3 + 3 * nbuf]
        cid = lax.axis_index("c")
        sid = lax.axis_index("s")
        wid = cid * _NS + sid
        base = wid * ipw

        pltpu.async_copy(iu_hbm.at[pl.ds(base, ipw)], iu_v, sem_m)
        pltpu.async_copy(is_hbm.at[pl.ds(base, ipw)], is_v, sem_m)
        pltpu.make_async_copy(iu_hbm.at[pl.ds(base, ipw)], iu_v, sem_m).wait()
        pltpu.make_async_copy(is_hbm.at[pl.ds(base, ipw)], is_v, sem_m).wait()

        def chunk(k_):
            side = k_ // (nch // 2)
            j = k_ % (nch // 2)
            t = (tu_hbm, ts_hbm)[side]
            o = (ou_hbm, os_hbm)[side]
            iv = (iu_v, is_v)[side]
            idx = iv.at[pl.ds(j * gch, gch)]
            return t.at[idx], o.at[pl.ds(base + j * gch, gch)]

        for k_ in range(nbuf):
            src, _ = chunk(k_)
            pltpu.async_copy(src, bufs[k_], sem_g[k_])
        for k_ in range(nch):
            b = k_ % nbuf
            src, dst = chunk(k_)
            pltpu.make_async_copy(src, bufs[b], sem_g[b]).wait()
            pltpu.async_copy(bufs[b], dst, sem_w[b])
            if k_ + nbuf < nch:
                pltpu.make_async_copy(bufs[b], dst, sem_w[b]).wait()
                nsrc, _ = chunk(k_ + nbuf)
                pltpu.async_copy(nsrc, bufs[b], sem_g[b])
        for k_ in range(nch - nbuf, nch):
            b = k_ % nbuf
            _, dst = chunk(k_)
            pltpu.make_async_copy(bufs[b], dst, sem_w[b]).wait()

    return k(tab_u, idx_u, tab_s, idx_s)


_LN_EPS = 1e-5


def _dense_body(acc_ref, deg_ref, feats_ref, w_ref, b_ref, g_ref, beta_ref,
                out_ref):
    agg = acc_ref[0] + acc_ref[1] + feats_ref[...]
    deg = deg_ref[0] + deg_ref[1] + 1.0
    h = agg / deg
    h = jnp.dot(h, w_ref[...], preferred_element_type=jnp.float32) + b_ref[...]
    mu = jnp.mean(h, axis=-1, keepdims=True)
    var = jnp.mean((h - mu) ** 2, axis=-1, keepdims=True)
    h = (h - mu) * lax.rsqrt(var + _LN_EPS) * g_ref[...] + beta_ref[...]
    out_ref[...] = jnp.where(h > 0, h, jnp.exp(jnp.minimum(h, 0.0)) - 1.0)


def _dense_tc(acc, deg, feats, w, b, g, beta, interpret=False):
    r = 400
    return pl.pallas_call(
        _dense_body,
        grid=(_N // r,),
        in_specs=[
            pl.BlockSpec((_NC, r, _D), lambda i: (0, i, 0)),
            pl.BlockSpec((_NC, r, 1), lambda i: (0, i, 0)),
            pl.BlockSpec((r, _D), lambda i: (i, 0)),
            pl.BlockSpec((_D, _D), lambda i: (0, 0)),
            pl.BlockSpec((1, _D), lambda i: (0, 0)),
            pl.BlockSpec((1, _D), lambda i: (0, 0)),
            pl.BlockSpec((1, _D), lambda i: (0, 0)),
        ],
        out_specs=pl.BlockSpec((r, _D), lambda i: (i, 0)),
        out_shape=jax.ShapeDtypeStruct((_N, _D), jnp.float32),
        interpret=interpret,
    )(acc, deg[..., None], feats, w, b.reshape(1, _D), g.reshape(1, _D),
      beta.reshape(1, _D))


def _mlp_body(u_ref, s_ref, w1u_ref, w1s_ref, b1_ref, g1_ref, beta1_ref,
              w2_ref, b2_ref, g2_ref, beta2_ref, w3_ref, b3_ref, out_ref):
    h = (jnp.dot(u_ref[...], w1u_ref[...], preferred_element_type=jnp.float32)
         + jnp.dot(s_ref[...], w1s_ref[...], preferred_element_type=jnp.float32)
         + b1_ref[...])
    mu = jnp.mean(h, axis=-1, keepdims=True)
    var = jnp.mean((h - mu) ** 2, axis=-1, keepdims=True)
    h = (h - mu) * lax.rsqrt(var + _LN_EPS) * g1_ref[...] + beta1_ref[...]
    h = jnp.maximum(h, 0.0)
    h = jnp.dot(h, w2_ref[...], preferred_element_type=jnp.float32) + b2_ref[...]
    mu = jnp.mean(h, axis=-1, keepdims=True)
    var = jnp.mean((h - mu) ** 2, axis=-1, keepdims=True)
    h = (h - mu) * lax.rsqrt(var + _LN_EPS) * g2_ref[...] + beta2_ref[...]
    h = jnp.maximum(h, 0.0)
    z = jnp.dot(h, w3_ref[...], preferred_element_type=jnp.float32) + b3_ref[...]
    out_ref[...] = jax.nn.sigmoid(z)


def _mlp_tc(u, s, w1, b1, g1, beta1, w2, b2, g2, beta2, w3, b3,
            interpret=False):
    r = 2048
    hid = w2.shape[0]
    return pl.pallas_call(
        _mlp_body,
        grid=(_B // r,),
        in_specs=[
            pl.BlockSpec((r, _D), lambda i: (i, 0)),
            pl.BlockSpec((r, _D), lambda i: (i, 0)),
            pl.BlockSpec((_D, hid), lambda i: (0, 0)),
            pl.BlockSpec((_D, hid), lambda i: (0, 0)),
            pl.BlockSpec((1, hid), lambda i: (0, 0)),
            pl.BlockSpec((1, hid), lambda i: (0, 0)),
            pl.BlockSpec((1, hid), lambda i: (0, 0)),
            pl.BlockSpec((hid, hid), lambda i: (0, 0)),
            pl.BlockSpec((1, hid), lambda i: (0, 0)),
            pl.BlockSpec((1, hid), lambda i: (0, 0)),
            pl.BlockSpec((1, hid), lambda i: (0, 0)),
            pl.BlockSpec((hid, 1), lambda i: (0, 0)),
            pl.BlockSpec((1, 1), lambda i: (0, 0)),
        ],
        out_specs=pl.BlockSpec((r, 1), lambda i: (i, 0)),
        out_shape=jax.ShapeDtypeStruct((_B, 1), jnp.float32),
        interpret=interpret,
    )(u, s, w1[:_D], w1[_D:], b1.reshape(1, -1), g1.reshape(1, -1),
      beta1.reshape(1, -1), w2, b2.reshape(1, -1), g2.reshape(1, -1),
      beta2.reshape(1, -1), w3, b3.reshape(1, 1))


def kernel(params, user_edges, serv_edges, userIdx, itemIdx):
    p = params
    e = {"user": (user_edges[0].reshape(_NROW, _CH),
                  user_edges[1].reshape(_NROW, _CH)),
         "serv": (serv_edges[0].reshape(_NROW, _CH),
                  serv_edges[1].reshape(_NROW, _CH))}
    feats = {"user": p["user_emb"], "serv": p["serv_emb"]}
    deg = {}

    def dense(side, l, acc):
        return _dense_tc(acc, deg[side], feats[side], p[f"{side}_W{l}"],
                         p[f"{side}_b{l}"], p[f"{side}_g{l}"],
                         p[f"{side}_beta{l}"])

    acc_u, d = _seg_sum_sc(feats["user"], *e["user"], True)
    deg["user"] = d.reshape(_NC, _NP)
    acc_s, d = _seg_sum_sc(feats["serv"], *e["serv"], True)
    deg["serv"] = d.reshape(_NC, _NP)
    feats["user"] = dense("user", 0, acc_u)
    (acc_u,) = _seg_sum_sc(feats["user"], *e["user"], False)
    feats["serv"] = dense("serv", 0, acc_s)
    (acc_s,) = _seg_sum_sc(feats["serv"], *e["serv"], False)
    feats["user"] = dense("user", 1, acc_u)
    feats["serv"] = dense("serv", 1, acc_s)
    u, s = _gather2_sc(feats["user"], userIdx, feats["serv"], itemIdx)
    est = _mlp_tc(u, s, p["W1"], p["b1"], p["g1"], p["beta1"], p["W2"],
                  p["b2"], p["g2"], p["beta2"], p["W3"], p["b3"])
    return est.reshape(_B)


# speedup vs baseline: 7.4701x; 1.0041x over previous
import functools

import jax
import jax.numpy as jnp
from jax import lax
from jax.experimental import pallas as pl
from jax.experimental.pallas import tpu as pltpu
from jax.experimental.pallas import tpu_sc as plsc

_N = 10000
_D = 128
_E = 320000
_B = 16384
_NC = 2
_NS = 16
_NW = _NC * _NS
_CH = 125
_NCH = 80
_NROW = _E // _CH
_NP = 10240
_RPT = _NP // _NS
_ZB = 8
_NB = 2
_IB = 40


def _seg_sum_sc(feats, src2d, dst2d, with_deg):
    mesh = plsc.VectorSubcoreMesh(core_axis_name="c", subcore_axis_name="s")

    out_type = [jax.ShapeDtypeStruct((_NC, _NP, _D), jnp.float32)]
    if with_deg:
        out_type.append(jax.ShapeDtypeStruct((_NC * _NP,), jnp.float32))

    scratch = [
        pltpu.VMEM((_IB, _CH), jnp.int32),
        pltpu.VMEM((_IB, _CH), jnp.int32),
        pltpu.VMEM((_CH, _D), jnp.float32),
        pltpu.VMEM((_CH, _D), jnp.float32),
        pltpu.VMEM((128,), jnp.float32),
        pltpu.VMEM((_ZB, _D), jnp.float32),
        pltpu.VMEM((_RPT,), jnp.float32),
        pltpu.VMEM_SHARED((_NP, _D), jnp.float32),
        pltpu.VMEM_SHARED((_NP,), jnp.float32),
        pltpu.SemaphoreType.DMA,
        pltpu.SemaphoreType.DMA,
        pltpu.SemaphoreType.DMA,
        pltpu.SemaphoreType.DMA,
        pltpu.SemaphoreType.DMA,
        pltpu.SemaphoreType.DMA,
    ]

    @functools.partial(pl.kernel, out_type=tuple(out_type), mesh=mesh,
                       scratch_types=scratch)
    def k(feats_hbm, src_hbm, dst_hbm, acc_hbm, *rest):
        if with_deg:
            deg_hbm = rest[0]
            rest = rest[1:]
        (srcs_v, dsts_v, rows_a, rows_b, ones_v, zrow_v, zdeg_v,
         acc_sh, deg_sh, sem_m, sem_ga, sem_gb, sem_sa, sem_sb,
         sem_d) = rest
        cid = lax.axis_index("c")
        sid = lax.axis_index("s")
        wid = cid * _NS + sid
        bufs = ((rows_a, sem_ga, sem_sa), (rows_b, sem_gb, sem_sb))

        z16 = jnp.zeros((16,), jnp.float32)

        @pl.loop(0, _ZB)
        def _(r):
            @pl.loop(0, _D, step=16)
            def _(c):
                zrow_v[r, pl.ds(c, 16)] = z16

        if with_deg:
            o16 = jnp.ones((16,), jnp.float32)

            @pl.loop(0, _RPT, step=16)
            def _(i):
                zdeg_v[pl.ds(i, 16)] = z16

            @pl.loop(0, 128, step=16)
            def _(i):
                ones_v[pl.ds(i, 16)] = o16

        for i in range(_RPT // _ZB):
            pltpu.async_copy(
                zrow_v, acc_sh.at[pl.ds(sid * _RPT + i * _ZB, _ZB)], sem_m)
        if with_deg:
            pltpu.async_copy(zdeg_v, deg_sh.at[pl.ds(sid * _RPT, _RPT)],
                             sem_m)
        for i in range(_RPT // _ZB):
            pltpu.make_async_copy(
                zrow_v, acc_sh.at[pl.ds(sid * _RPT + i * _ZB, _ZB)],
                sem_m).wait()
        if with_deg:
            pltpu.make_async_copy(zdeg_v, deg_sh.at[pl.ds(sid * _RPT, _RPT)],
                                  sem_m).wait()

        plsc.subcore_barrier()

        @pl.loop(0, _NCH, step=_IB)
        def _(t0):
            row0 = wid * _NCH + t0
            pltpu.sync_copy(src_hbm.at[pl.ds(row0, _IB)], srcs_v)
            pltpu.sync_copy(dst_hbm.at[pl.ds(row0, _IB)], dsts_v)

            for b, (buf, sem_g, _) in enumerate(bufs):
                pltpu.async_copy(feats_hbm.at[srcs_v.at[b]], buf, sem_g)

            @pl.loop(0, _IB, step=_NB)
            def _(t):
                @pl.when(t > 0)
                def _():
                    for b, (buf, sem_g, sem_s) in enumerate(bufs):
                        j = t + b
                        pltpu.make_async_copy(
                            buf, acc_sh.at[dsts_v.at[j - _NB]], sem_s).wait()
                        pltpu.async_copy(feats_hbm.at[srcs_v.at[j]], buf,
                                         sem_g)

                for b, (buf, sem_g, sem_s) in enumerate(bufs):
                    j = t + b
                    pltpu.make_async_copy(feats_hbm.at[srcs_v.at[j]], buf,
                                          sem_g).wait()
                    pltpu.async_copy(buf, acc_sh.at[dsts_v.at[j]], sem_s,
                                     add=True)
                    if with_deg:
                        pltpu.async_copy(ones_v.at[pl.ds(0, _CH)],
                                         deg_sh.at[dsts_v.at[j]], sem_d,
                                         add=True)

            for b, (buf, sem_g, sem_s) in enumerate(bufs):
                j = _IB - _NB + b
                pltpu.make_async_copy(buf, acc_sh.at[dsts_v.at[j]],
                                      sem_s).wait()

            if with_deg:
                @pl.loop(0, _IB)
                def _(j):
                    pltpu.make_async_copy(ones_v.at[pl.ds(0, _CH)],
                                          deg_sh.at[dsts_v.at[j]],
                                          sem_d).wait()

        plsc.subcore_barrier()

        pltpu.sync_copy(acc_sh.at[pl.ds(sid * _RPT, _RPT)],
                        acc_hbm.at[cid, pl.ds(sid * _RPT, _RPT)])
        if with_deg:
            pltpu.sync_copy(deg_sh.at[pl.ds(sid * _RPT, _RPT)],
                            deg_hbm.at[pl.ds(cid * _NP + sid * _RPT, _RPT)])

    return k(feats, src2d, dst2d)


def _gather2_sc(tab_u, idx_u, tab_s, idx_s):
    mesh = plsc.VectorSubcoreMesh(core_axis_name="c", subcore_axis_name="s")
    ipw = _B // _NW
    gch = 128
    nbuf = 6
    nch = 2 * (ipw // gch)

    scratch = ([pltpu.VMEM((ipw,), jnp.int32)] * 2 +
               [pltpu.VMEM((gch, _D), jnp.float32)] * nbuf +
               [pltpu.SemaphoreType.DMA] * (1 + 2 * nbuf))

    @functools.partial(
        pl.kernel,
        out_type=(jax.ShapeDtypeStruct((_B, _D), jnp.float32),
                  jax.ShapeDtypeStruct((_B, _D), jnp.float32)),
        mesh=mesh,
        scratch_types=scratch,
    )
    def k(tu_hbm, iu_hbm, ts_hbm, is_hbm, ou_hbm, os_hbm, *rest):
        iu_v, is_v = rest[0], rest[1]
        bufs = rest[2:2 + nbuf]
        sem_m = rest[2 + nbuf]
        sem_g = rest[3 + nbuf:3 + 2 * nbuf]
        sem_w = rest[3 + 2 * nbuf:3 + 3 * nbuf]
        cid = lax.axis_index("c")
        sid = lax.axis_index("s")
        wid = cid * _NS + sid
        base = wid * ipw

        pltpu.async_copy(iu_hbm.at[pl.ds(base, ipw)], iu_v, sem_m)
        pltpu.async_copy(is_hbm.at[pl.ds(base, ipw)], is_v, sem_m)
        pltpu.make_async_copy(iu_hbm.at[pl.ds(base, ipw)], iu_v, sem_m).wait()
        pltpu.make_async_copy(is_hbm.at[pl.ds(base, ipw)], is_v, sem_m).wait()

        def chunk(k_):
            side = k_ // (nch // 2)
            j = k_ % (nch // 2)
            t = (tu_hbm, ts_hbm)[side]
            o = (ou_hbm, os_hbm)[side]
            iv = (iu_v, is_v)[side]
            idx = iv.at[pl.ds(j * gch, gch)]
            return t.at[idx], o.at[pl.ds(base + j * gch, gch)]

        for k_ in range(nbuf):
            src, _ = chunk(k_)
            pltpu.async_copy(src, bufs[k_], sem_g[k_])
        for k_ in range(nch):
            b = k_ % nbuf
            src, dst = chunk(k_)
            pltpu.make_async_copy(src, bufs[b], sem_g[b]).wait()
            pltpu.async_copy(bufs[b], dst, sem_w[b])
            if k_ + nbuf < nch:
                pltpu.make_async_copy(bufs[b], dst, sem_w[b]).wait()
                nsrc, _ = chunk(k_ + nbuf)
                pltpu.async_copy(nsrc, bufs[b], sem_g[b])
        for k_ in range(nch - nbuf, nch):
            b = k_ % nbuf
            _, dst = chunk(k_)
            pltpu.make_async_copy(bufs[b], dst, sem_w[b]).wait()

    return k(tab_u, idx_u, tab_s, idx_s)


_LN_EPS = 1e-5


def _dense_body(acc_ref, deg_ref, feats_ref, w_ref, b_ref, g_ref, beta_ref,
                out_ref):
    agg = acc_ref[0] + acc_ref[1] + feats_ref[...]
    deg = deg_ref[0] + deg_ref[1] + 1.0
    h = agg / deg
    h = jnp.dot(h, w_ref[...], preferred_element_type=jnp.float32) + b_ref[...]
    mu = jnp.mean(h, axis=-1, keepdims=True)
    var = jnp.mean((h - mu) ** 2, axis=-1, keepdims=True)
    h = (h - mu) * lax.rsqrt(var + _LN_EPS) * g_ref[...] + beta_ref[...]
    out_ref[...] = jnp.where(h > 0, h, jnp.exp(jnp.minimum(h, 0.0)) - 1.0)


def _dense_tc(acc, deg, feats, w, b, g, beta, interpret=False):
    r = 400
    return pl.pallas_call(
        _dense_body,
        grid=(_N // r,),
        in_specs=[
            pl.BlockSpec((_NC, r, _D), lambda i: (0, i, 0)),
            pl.BlockSpec((_NC, r, 1), lambda i: (0, i, 0)),
            pl.BlockSpec((r, _D), lambda i: (i, 0)),
            pl.BlockSpec((_D, _D), lambda i: (0, 0)),
            pl.BlockSpec((1, _D), lambda i: (0, 0)),
            pl.BlockSpec((1, _D), lambda i: (0, 0)),
            pl.BlockSpec((1, _D), lambda i: (0, 0)),
        ],
        out_specs=pl.BlockSpec((r, _D), lambda i: (i, 0)),
        out_shape=jax.ShapeDtypeStruct((_N, _D), jnp.float32),
        interpret=interpret,
    )(acc, deg[..., None], feats, w, b.reshape(1, _D), g.reshape(1, _D),
      beta.reshape(1, _D))


def _mlp_body(u_ref, s_ref, w1u_ref, w1s_ref, b1_ref, g1_ref, beta1_ref,
              w2_ref, b2_ref, g2_ref, beta2_ref, w3_ref, b3_ref, out_ref):
    h = (jnp.dot(u_ref[...], w1u_ref[...], preferred_element_type=jnp.float32)
         + jnp.dot(s_ref[...], w1s_ref[...], preferred_element_type=jnp.float32)
         + b1_ref[...])
    mu = jnp.mean(h, axis=-1, keepdims=True)
    var = jnp.mean((h - mu) ** 2, axis=-1, keepdims=True)
    h = (h - mu) * lax.rsqrt(var + _LN_EPS) * g1_ref[...] + beta1_ref[...]
    h = jnp.maximum(h, 0.0)
    h = jnp.dot(h, w2_ref[...], preferred_element_type=jnp.float32) + b2_ref[...]
    mu = jnp.mean(h, axis=-1, keepdims=True)
    var = jnp.mean((h - mu) ** 2, axis=-1, keepdims=True)
    h = (h - mu) * lax.rsqrt(var + _LN_EPS) * g2_ref[...] + beta2_ref[...]
    h = jnp.maximum(h, 0.0)
    z = jnp.dot(h, w3_ref[...], preferred_element_type=jnp.float32) + b3_ref[...]
    out_ref[...] = jax.nn.sigmoid(z)


def _mlp_tc(u, s, w1, b1, g1, beta1, w2, b2, g2, beta2, w3, b3,
            interpret=False):
    r = 2048
    hid = w2.shape[0]
    return pl.pallas_call(
        _mlp_body,
        grid=(_B // r,),
        in_specs=[
            pl.BlockSpec((r, _D), lambda i: (i, 0)),
            pl.BlockSpec((r, _D), lambda i: (i, 0)),
            pl.BlockSpec((_D, hid), lambda i: (0, 0)),
            pl.BlockSpec((_D, hid), lambda i: (0, 0)),
            pl.BlockSpec((1, hid), lambda i: (0, 0)),
            pl.BlockSpec((1, hid), lambda i: (0, 0)),
            pl.BlockSpec((1, hid), lambda i: (0, 0)),
            pl.BlockSpec((hid, hid), lambda i: (0, 0)),
            pl.BlockSpec((1, hid), lambda i: (0, 0)),
            pl.BlockSpec((1, hid), lambda i: (0, 0)),
            pl.BlockSpec((1, hid), lambda i: (0, 0)),
            pl.BlockSpec((hid, 1), lambda i: (0, 0)),
            pl.BlockSpec((1, 1), lambda i: (0, 0)),
        ],
        out_specs=pl.BlockSpec((r, 1), lambda i: (i, 0)),
        out_shape=jax.ShapeDtypeStruct((_B, 1), jnp.float32),
        interpret=interpret,
    )(u, s, w1[:_D], w1[_D:], b1.reshape(1, -1), g1.reshape(1, -1),
      beta1.reshape(1, -1), w2, b2.reshape(1, -1), g2.reshape(1, -1),
      beta2.reshape(1, -1), w3, b3.reshape(1, 1))


def kernel(params, user_edges, serv_edges, userIdx, itemIdx):
    p = params
    e = {"user": (user_edges[0].reshape(_NROW, _CH),
                  user_edges[1].reshape(_NROW, _CH)),
         "serv": (serv_edges[0].reshape(_NROW, _CH),
                  serv_edges[1].reshape(_NROW, _CH))}
    feats = {"user": p["user_emb"], "serv": p["serv_emb"]}
    deg = {}

    def dense(side, l, acc):
        return _dense_tc(acc, deg[side], feats[side], p[f"{side}_W{l}"],
                         p[f"{side}_b{l}"], p[f"{side}_g{l}"],
                         p[f"{side}_beta{l}"])

    acc_u, d = _seg_sum_sc(feats["user"], *e["user"], True)
    deg["user"] = d.reshape(_NC, _NP)
    acc_s, d = _seg_sum_sc(feats["serv"], *e["serv"], True)
    deg["serv"] = d.reshape(_NC, _NP)
    feats["user"] = dense("user", 0, acc_u)
    (acc_u,) = _seg_sum_sc(feats["user"], *e["user"], False)
    feats["serv"] = dense("serv", 0, acc_s)
    (acc_s,) = _seg_sum_sc(feats["serv"], *e["serv"], False)
    feats["user"] = dense("user", 1, acc_u)
    feats["serv"] = dense("serv", 1, acc_s)
    u, s = _gather2_sc(feats["user"], userIdx, feats["serv"], itemIdx)
    est = _mlp_tc(u, s, p["W1"], p["b1"], p["g1"], p["beta1"], p["W2"],
                  p["b2"], p["g2"], p["beta2"], p["W3"], p["b3"])
    return est.reshape(_B)


# speedup vs baseline: 7.4763x; 1.0008x over previous
import functools

import jax
import jax.numpy as jnp
from jax import lax
from jax.experimental import pallas as pl
from jax.experimental.pallas import tpu as pltpu
from jax.experimental.pallas import tpu_sc as plsc

_N = 10000
_D = 128
_E = 320000
_B = 16384
_NC = 2
_NS = 16
_NW = _NC * _NS
_CH = 125
_NCH = 80
_NROW = _E // _CH
_NP = 10240
_RPT = _NP // _NS
_ZB = 32
_NB = 2
_IB = 40


def _seg_sum_sc(feats, src2d, dst2d, with_deg):
    mesh = plsc.VectorSubcoreMesh(core_axis_name="c", subcore_axis_name="s")

    out_type = [jax.ShapeDtypeStruct((_NC, _NP, _D), jnp.float32)]
    if with_deg:
        out_type.append(jax.ShapeDtypeStruct((_NC * _NP,), jnp.float32))

    scratch = [
        pltpu.VMEM((_IB, _CH), jnp.int32),
        pltpu.VMEM((_IB, _CH), jnp.int32),
        pltpu.VMEM((_CH, _D), jnp.float32),
        pltpu.VMEM((_CH, _D), jnp.float32),
        pltpu.VMEM((128,), jnp.float32),
        pltpu.VMEM((_ZB, _D), jnp.float32),
        pltpu.VMEM((_RPT,), jnp.float32),
        pltpu.VMEM_SHARED((_NP, _D), jnp.float32),
        pltpu.VMEM_SHARED((_NP,), jnp.float32),
        pltpu.SemaphoreType.DMA,
        pltpu.SemaphoreType.DMA,
        pltpu.SemaphoreType.DMA,
        pltpu.SemaphoreType.DMA,
        pltpu.SemaphoreType.DMA,
        pltpu.SemaphoreType.DMA,
    ]

    @functools.partial(pl.kernel, out_type=tuple(out_type), mesh=mesh,
                       scratch_types=scratch)
    def k(feats_hbm, src_hbm, dst_hbm, acc_hbm, *rest):
        if with_deg:
            deg_hbm = rest[0]
            rest = rest[1:]
        (srcs_v, dsts_v, rows_a, rows_b, ones_v, zrow_v, zdeg_v,
         acc_sh, deg_sh, sem_m, sem_ga, sem_gb, sem_sa, sem_sb,
         sem_d) = rest
        cid = lax.axis_index("c")
        sid = lax.axis_index("s")
        wid = cid * _NS + sid
        bufs = ((rows_a, sem_ga, sem_sa), (rows_b, sem_gb, sem_sb))

        z16 = jnp.zeros((16,), jnp.float32)

        @pl.loop(0, _ZB)
        def _(r):
            @pl.loop(0, _D, step=16)
            def _(c):
                zrow_v[r, pl.ds(c, 16)] = z16

        if with_deg:
            o16 = jnp.ones((16,), jnp.float32)

            @pl.loop(0, _RPT, step=16)
            def _(i):
                zdeg_v[pl.ds(i, 16)] = z16

            @pl.loop(0, 128, step=16)
            def _(i):
                ones_v[pl.ds(i, 16)] = o16

        for i in range(_RPT // _ZB):
            pltpu.async_copy(
                zrow_v, acc_sh.at[pl.ds(sid * _RPT + i * _ZB, _ZB)], sem_m)
        if with_deg:
            pltpu.async_copy(zdeg_v, deg_sh.at[pl.ds(sid * _RPT, _RPT)],
                             sem_m)
        for i in range(_RPT // _ZB):
            pltpu.make_async_copy(
                zrow_v, acc_sh.at[pl.ds(sid * _RPT + i * _ZB, _ZB)],
                sem_m).wait()
        if with_deg:
            pltpu.make_async_copy(zdeg_v, deg_sh.at[pl.ds(sid * _RPT, _RPT)],
                                  sem_m).wait()

        plsc.subcore_barrier()

        @pl.loop(0, _NCH, step=_IB)
        def _(t0):
            row0 = wid * _NCH + t0
            pltpu.sync_copy(src_hbm.at[pl.ds(row0, _IB)], srcs_v)
            pltpu.sync_copy(dst_hbm.at[pl.ds(row0, _IB)], dsts_v)

            for b, (buf, sem_g, _) in enumerate(bufs):
                pltpu.async_copy(feats_hbm.at[srcs_v.at[b]], buf, sem_g)

            @pl.loop(0, _IB, step=_NB)
            def _(t):
                for b, (buf, sem_g, sem_s) in enumerate(bufs):
                    j = t + b
                    pltpu.make_async_copy(feats_hbm.at[srcs_v.at[j]], buf,
                                          sem_g).wait()
                    pltpu.async_copy(buf, acc_sh.at[dsts_v.at[j]], sem_s,
                                     add=True)
                    if with_deg:
                        pltpu.async_copy(ones_v.at[pl.ds(0, _CH)],
                                         deg_sh.at[dsts_v.at[j]], sem_d,
                                         add=True)
                for b, (buf, sem_g, sem_s) in enumerate(bufs):
                    j = t + b
                    pltpu.make_async_copy(buf, acc_sh.at[dsts_v.at[j]],
                                          sem_s).wait()

                    @pl.when(t + _NB < _IB)
                    def _(j=j, buf=buf, sem_g=sem_g):
                        pltpu.async_copy(feats_hbm.at[srcs_v.at[j + _NB]],
                                         buf, sem_g)

            if with_deg:
                @pl.loop(0, _IB)
                def _(j):
                    pltpu.make_async_copy(ones_v.at[pl.ds(0, _CH)],
                                          deg_sh.at[dsts_v.at[j]],
                                          sem_d).wait()

        plsc.subcore_barrier()

        pltpu.sync_copy(acc_sh.at[pl.ds(sid * _RPT, _RPT)],
                        acc_hbm.at[cid, pl.ds(sid * _RPT, _RPT)])
        if with_deg:
            pltpu.sync_copy(deg_sh.at[pl.ds(sid * _RPT, _RPT)],
                            deg_hbm.at[pl.ds(cid * _NP + sid * _RPT, _RPT)])

    return k(feats, src2d, dst2d)


def _gather2_sc(tab_u, idx_u, tab_s, idx_s):
    mesh = plsc.VectorSubcoreMesh(core_axis_name="c", subcore_axis_name="s")
    ipw = _B // _NW
    gch = 128
    nbuf = 6
    nch = 2 * (ipw // gch)

    scratch = ([pltpu.VMEM((ipw,), jnp.int32)] * 2 +
               [pltpu.VMEM((gch, _D), jnp.float32)] * nbuf +
               [pltpu.SemaphoreType.DMA] * (1 + 2 * nbuf))

    @functools.partial(
        pl.kernel,
        out_type=(jax.ShapeDtypeStruct((_B, _D), jnp.float32),
                  jax.ShapeDtypeStruct((_B, _D), jnp.float32)),
        mesh=mesh,
        scratch_types=scratch,
    )
    def k(tu_hbm, iu_hbm, ts_hbm, is_hbm, ou_hbm, os_hbm, *rest):
        iu_v, is_v = rest[0], rest[1]
        bufs = rest[2:2 + nbuf]
        sem_m = rest[2 + nbuf]
        sem_g = rest[3 + nbuf:3 + 2 * nbuf]
        sem_w = rest[3 + 2 * nbuf:3 + 3 * nbuf]
        cid = lax.axis_index("c")
        sid = lax.axis_index("s")
        wid = cid * _NS + sid
        base = wid * ipw

        pltpu.async_copy(iu_hbm.at[pl.ds(base, ipw)], iu_v, sem_m)
        pltpu.async_copy(is_hbm.at[pl.ds(base, ipw)], is_v, sem_m)
        pltpu.make_async_copy(iu_hbm.at[pl.ds(base, ipw)], iu_v, sem_m).wait()
        pltpu.make_async_copy(is_hbm.at[pl.ds(base, ipw)], is_v, sem_m).wait()

        def chunk(k_):
            side = k_ // (nch // 2)
            j = k_ % (nch // 2)
            t = (tu_hbm, ts_hbm)[side]
            o = (ou_hbm, os_hbm)[side]
            iv = (iu_v, is_v)[side]
            idx = iv.at[pl.ds(j * gch, gch)]
            return t.at[idx], o.at[pl.ds(base + j * gch, gch)]

        for k_ in range(nbuf):
            src, _ = chunk(k_)
            pltpu.async_copy(src, bufs[k_], sem_g[k_])
        for k_ in range(nch):
            b = k_ % nbuf
            src, dst = chunk(k_)
            pltpu.make_async_copy(src, bufs[b], sem_g[b]).wait()
            pltpu.async_copy(bufs[b], dst, sem_w[b])
            if k_ + nbuf < nch:
                pltpu.make_async_copy(bufs[b], dst, sem_w[b]).wait()
                nsrc, _ = chunk(k_ + nbuf)
                pltpu.async_copy(nsrc, bufs[b], sem_g[b])
        for k_ in range(nch - nbuf, nch):
            b = k_ % nbuf
            _, dst = chunk(k_)
            pltpu.make_async_copy(bufs[b], dst, sem_w[b]).wait()

    return k(tab_u, idx_u, tab_s, idx_s)


_LN_EPS = 1e-5


def _dense_body(acc_ref, deg_ref, feats_ref, w_ref, b_ref, g_ref, beta_ref,
                out_ref):
    agg = acc_ref[0] + acc_ref[1] + feats_ref[...]
    deg = deg_ref[0] + deg_ref[1] + 1.0
    h = agg / deg
    h = jnp.dot(h, w_ref[...], preferred_element_type=jnp.float32) + b_ref[...]
    mu = jnp.mean(h, axis=-1, keepdims=True)
    var = jnp.mean((h - mu) ** 2, axis=-1, keepdims=True)
    h = (h - mu) * lax.rsqrt(var + _LN_EPS) * g_ref[...] + beta_ref[...]
    out_ref[...] = jnp.where(h > 0, h, jnp.exp(jnp.minimum(h, 0.0)) - 1.0)


def _dense_tc(acc, deg, feats, w, b, g, beta, interpret=False):
    r = 400
    return pl.pallas_call(
        _dense_body,
        grid=(_N // r,),
        in_specs=[
            pl.BlockSpec((_NC, r, _D), lambda i: (0, i, 0)),
            pl.BlockSpec((_NC, r, 1), lambda i: (0, i, 0)),
            pl.BlockSpec((r, _D), lambda i: (i, 0)),
            pl.BlockSpec((_D, _D), lambda i: (0, 0)),
            pl.BlockSpec((1, _D), lambda i: (0, 0)),
            pl.BlockSpec((1, _D), lambda i: (0, 0)),
            pl.BlockSpec((1, _D), lambda i: (0, 0)),
        ],
        out_specs=pl.BlockSpec((r, _D), lambda i: (i, 0)),
        out_shape=jax.ShapeDtypeStruct((_N, _D), jnp.float32),
        interpret=interpret,
    )(acc, deg[..., None], feats, w, b.reshape(1, _D), g.reshape(1, _D),
      beta.reshape(1, _D))


def _mlp_body(u_ref, s_ref, w1u_ref, w1s_ref, b1_ref, g1_ref, beta1_ref,
              w2_ref, b2_ref, g2_ref, beta2_ref, w3_ref, b3_ref, out_ref):
    h = (jnp.dot(u_ref[...], w1u_ref[...], preferred_element_type=jnp.float32)
         + jnp.dot(s_ref[...], w1s_ref[...], preferred_element_type=jnp.float32)
         + b1_ref[...])
    mu = jnp.mean(h, axis=-1, keepdims=True)
    var = jnp.mean((h - mu) ** 2, axis=-1, keepdims=True)
    h = (h - mu) * lax.rsqrt(var + _LN_EPS) * g1_ref[...] + beta1_ref[...]
    h = jnp.maximum(h, 0.0)
    h = jnp.dot(h, w2_ref[...], preferred_element_type=jnp.float32) + b2_ref[...]
    mu = jnp.mean(h, axis=-1, keepdims=True)
    var = jnp.mean((h - mu) ** 2, axis=-1, keepdims=True)
    h = (h - mu) * lax.rsqrt(var + _LN_EPS) * g2_ref[...] + beta2_ref[...]
    h = jnp.maximum(h, 0.0)
    z = jnp.dot(h, w3_ref[...], preferred_element_type=jnp.float32) + b3_ref[...]
    out_ref[...] = jax.nn.sigmoid(z)


def _mlp_tc(u, s, w1, b1, g1, beta1, w2, b2, g2, beta2, w3, b3,
            interpret=False):
    r = 2048
    hid = w2.shape[0]
    return pl.pallas_call(
        _mlp_body,
        grid=(_B // r,),
        in_specs=[
            pl.BlockSpec((r, _D), lambda i: (i, 0)),
            pl.BlockSpec((r, _D), lambda i: (i, 0)),
            pl.BlockSpec((_D, hid), lambda i: (0, 0)),
            pl.BlockSpec((_D, hid), lambda i: (0, 0)),
            pl.BlockSpec((1, hid), lambda i: (0, 0)),
            pl.BlockSpec((1, hid), lambda i: (0, 0)),
            pl.BlockSpec((1, hid), lambda i: (0, 0)),
            pl.BlockSpec((hid, hid), lambda i: (0, 0)),
            pl.BlockSpec((1, hid), lambda i: (0, 0)),
            pl.BlockSpec((1, hid), lambda i: (0, 0)),
            pl.BlockSpec((1, hid), lambda i: (0, 0)),
            pl.BlockSpec((hid, 1), lambda i: (0, 0)),
            pl.BlockSpec((1, 1), lambda i: (0, 0)),
        ],
        out_specs=pl.BlockSpec((r, 1), lambda i: (i, 0)),
        out_shape=jax.ShapeDtypeStruct((_B, 1), jnp.float32),
        interpret=interpret,
    )(u, s, w1[:_D], w1[_D:], b1.reshape(1, -1), g1.reshape(1, -1),
      beta1.reshape(1, -1), w2, b2.reshape(1, -1), g2.reshape(1, -1),
      beta2.reshape(1, -1), w3, b3.reshape(1, 1))


def kernel(params, user_edges, serv_edges, userIdx, itemIdx):
    p = params
    e = {"user": (user_edges[0].reshape(_NROW, _CH),
                  user_edges[1].reshape(_NROW, _CH)),
         "serv": (serv_edges[0].reshape(_NROW, _CH),
                  serv_edges[1].reshape(_NROW, _CH))}
    feats = {"user": p["user_emb"], "serv": p["serv_emb"]}
    deg = {}

    def dense(side, l, acc):
        return _dense_tc(acc, deg[side], feats[side], p[f"{side}_W{l}"],
                         p[f"{side}_b{l}"], p[f"{side}_g{l}"],
                         p[f"{side}_beta{l}"])

    acc_u, d = _seg_sum_sc(feats["user"], *e["user"], True)
    deg["user"] = d.reshape(_NC, _NP)
    acc_s, d = _seg_sum_sc(feats["serv"], *e["serv"], True)
    deg["serv"] = d.reshape(_NC, _NP)
    feats["user"] = dense("user", 0, acc_u)
    (acc_u,) = _seg_sum_sc(feats["user"], *e["user"], False)
    feats["serv"] = dense("serv", 0, acc_s)
    (acc_s,) = _seg_sum_sc(feats["serv"], *e["serv"], False)
    feats["user"] = dense("user", 1, acc_u)
    feats["serv"] = dense("serv", 1, acc_s)
    u, s = _gather2_sc(feats["user"], userIdx, feats["serv"], itemIdx)
    est = _mlp_tc(u, s, p["W1"], p["b1"], p["g1"], p["beta1"], p["W2"],
                  p["b2"], p["g2"], p["beta2"], p["W3"], p["b3"])
    return est.reshape(_B)


# speedup vs baseline: 7.6651x; 1.0253x over previous
import functools

import jax
import jax.numpy as jnp
from jax import lax
from jax.experimental import pallas as pl
from jax.experimental.pallas import tpu as pltpu
from jax.experimental.pallas import tpu_sc as plsc

_N = 10000
_D = 128
_E = 320000
_B = 16384
_NC = 2
_NS = 16
_NW = _NC * _NS
_CH = 125
_NCH = 80
_NROW = _E // _CH
_NP = 10240
_RPT = _NP // _NS
_ZB = 32
_NB = 2
_IB = 40


def _seg_sum_sc(feats, src2d, dst2d, with_deg):
    mesh = plsc.VectorSubcoreMesh(core_axis_name="c", subcore_axis_name="s")

    out_type = [jax.ShapeDtypeStruct((_NC, _NP, _D), jnp.float32)]
    if with_deg:
        out_type.append(jax.ShapeDtypeStruct((_NC * _NP,), jnp.float32))

    scratch = [
        pltpu.VMEM((_IB, _CH), jnp.int32),
        pltpu.VMEM((_IB, _CH), jnp.int32),
        pltpu.VMEM((_CH, _D), jnp.float32),
        pltpu.VMEM((_CH, _D), jnp.float32),
        pltpu.VMEM((128,), jnp.float32),
        pltpu.VMEM((_ZB, _D), jnp.float32),
        pltpu.VMEM((_RPT,), jnp.float32),
        pltpu.VMEM_SHARED((_NP, _D), jnp.float32),
        pltpu.VMEM_SHARED((_NP,), jnp.float32),
        pltpu.SemaphoreType.DMA,
        pltpu.SemaphoreType.DMA,
        pltpu.SemaphoreType.DMA,
        pltpu.SemaphoreType.DMA,
        pltpu.SemaphoreType.DMA,
        pltpu.SemaphoreType.DMA,
    ]

    @functools.partial(pl.kernel, out_type=tuple(out_type), mesh=mesh,
                       scratch_types=scratch)
    def k(feats_hbm, src_hbm, dst_hbm, acc_hbm, *rest):
        if with_deg:
            deg_hbm = rest[0]
            rest = rest[1:]
        (srcs_v, dsts_v, rows_a, rows_b, ones_v, zrow_v, zdeg_v,
         acc_sh, deg_sh, sem_m, sem_ga, sem_gb, sem_sa, sem_sb,
         sem_d) = rest
        cid = lax.axis_index("c")
        sid = lax.axis_index("s")
        wid = cid * _NS + sid
        bufs = ((rows_a, sem_ga, sem_sa), (rows_b, sem_gb, sem_sb))

        pltpu.async_copy(src_hbm.at[pl.ds(wid * _NCH, _IB)], srcs_v, sem_d)
        pltpu.async_copy(dst_hbm.at[pl.ds(wid * _NCH, _IB)], dsts_v, sem_d)

        z16 = jnp.zeros((16,), jnp.float32)

        @pl.loop(0, _ZB)
        def _(r):
            @pl.loop(0, _D, step=16)
            def _(c):
                zrow_v[r, pl.ds(c, 16)] = z16

        if with_deg:
            o16 = jnp.ones((16,), jnp.float32)

            @pl.loop(0, _RPT, step=16)
            def _(i):
                zdeg_v[pl.ds(i, 16)] = z16

            @pl.loop(0, 128, step=16)
            def _(i):
                ones_v[pl.ds(i, 16)] = o16

        for i in range(_RPT // _ZB):
            pltpu.async_copy(
                zrow_v, acc_sh.at[pl.ds(sid * _RPT + i * _ZB, _ZB)], sem_m)
        if with_deg:
            pltpu.async_copy(zdeg_v, deg_sh.at[pl.ds(sid * _RPT, _RPT)],
                             sem_m)
        pltpu.make_async_copy(src_hbm.at[pl.ds(wid * _NCH, _IB)], srcs_v,
                              sem_d).wait()
        pltpu.make_async_copy(dst_hbm.at[pl.ds(wid * _NCH, _IB)], dsts_v,
                              sem_d).wait()
        for b, (buf, sem_g, _) in enumerate(bufs):
            pltpu.async_copy(feats_hbm.at[srcs_v.at[b]], buf, sem_g)

        for i in range(_RPT // _ZB):
            pltpu.make_async_copy(
                zrow_v, acc_sh.at[pl.ds(sid * _RPT + i * _ZB, _ZB)],
                sem_m).wait()
        if with_deg:
            pltpu.make_async_copy(zdeg_v, deg_sh.at[pl.ds(sid * _RPT, _RPT)],
                                  sem_m).wait()

        plsc.subcore_barrier()

        @pl.loop(0, _NCH, step=_IB)
        def _(t0):
            @pl.when(t0 > 0)
            def _():
                row0 = wid * _NCH + t0
                pltpu.sync_copy(src_hbm.at[pl.ds(row0, _IB)], srcs_v)
                pltpu.sync_copy(dst_hbm.at[pl.ds(row0, _IB)], dsts_v)
                for b, (buf, sem_g, _) in enumerate(bufs):
                    pltpu.async_copy(feats_hbm.at[srcs_v.at[b]], buf, sem_g)

            @pl.loop(0, _IB, step=_NB)
            def _(t):
                for b, (buf, sem_g, sem_s) in enumerate(bufs):
                    j = t + b
                    pltpu.make_async_copy(feats_hbm.at[srcs_v.at[j]], buf,
                                          sem_g).wait()
                    pltpu.async_copy(buf, acc_sh.at[dsts_v.at[j]], sem_s,
                                     add=True)
                    if with_deg:
                        pltpu.async_copy(ones_v.at[pl.ds(0, _CH)],
                                         deg_sh.at[dsts_v.at[j]], sem_d,
                                         add=True)
                for b, (buf, sem_g, sem_s) in enumerate(bufs):
                    j = t + b
                    pltpu.make_async_copy(buf, acc_sh.at[dsts_v.at[j]],
                                          sem_s).wait()

                    @pl.when(t + _NB < _IB)
                    def _(j=j, buf=buf, sem_g=sem_g):
                        pltpu.async_copy(feats_hbm.at[srcs_v.at[j + _NB]],
                                         buf, sem_g)

            if with_deg:
                @pl.loop(0, _IB)
                def _(j):
                    pltpu.make_async_copy(ones_v.at[pl.ds(0, _CH)],
                                          deg_sh.at[dsts_v.at[j]],
                                          sem_d).wait()

        plsc.subcore_barrier()

        pltpu.sync_copy(acc_sh.at[pl.ds(sid * _RPT, _RPT)],
                        acc_hbm.at[cid, pl.ds(sid * _RPT, _RPT)])
        if with_deg:
            pltpu.sync_copy(deg_sh.at[pl.ds(sid * _RPT, _RPT)],
                            deg_hbm.at[pl.ds(cid * _NP + sid * _RPT, _RPT)])

    return k(feats, src2d, dst2d)


def _gather2_sc(tab_u, idx_u, tab_s, idx_s):
    mesh = plsc.VectorSubcoreMesh(core_axis_name="c", subcore_axis_name="s")
    ipw = _B // _NW
    gch = 128
    nbuf = 6
    nch = 2 * (ipw // gch)

    scratch = ([pltpu.VMEM((ipw,), jnp.int32)] * 2 +
               [pltpu.VMEM((gch, _D), jnp.float32)] * nbuf +
               [pltpu.SemaphoreType.DMA] * (1 + 2 * nbuf))

    @functools.partial(
        pl.kernel,
        out_type=(jax.ShapeDtypeStruct((_B, _D), jnp.float32),
                  jax.ShapeDtypeStruct((_B, _D), jnp.float32)),
        mesh=mesh,
        scratch_types=scratch,
    )
    def k(tu_hbm, iu_hbm, ts_hbm, is_hbm, ou_hbm, os_hbm, *rest):
        iu_v, is_v = rest[0], rest[1]
        bufs = rest[2:2 + nbuf]
        sem_m = rest[2 + nbuf]
        sem_g = rest[3 + nbuf:3 + 2 * nbuf]
        sem_w = rest[3 + 2 * nbuf:3 + 3 * nbuf]
        cid = lax.axis_index("c")
        sid = lax.axis_index("s")
        wid = cid * _NS + sid
        base = wid * ipw

        pltpu.async_copy(iu_hbm.at[pl.ds(base, ipw)], iu_v, sem_m)
        pltpu.async_copy(is_hbm.at[pl.ds(base, ipw)], is_v, sem_m)
        pltpu.make_async_copy(iu_hbm.at[pl.ds(base, ipw)], iu_v, sem_m).wait()
        pltpu.make_async_copy(is_hbm.at[pl.ds(base, ipw)], is_v, sem_m).wait()

        def chunk(k_):
            side = k_ // (nch // 2)
            j = k_ % (nch // 2)
            t = (tu_hbm, ts_hbm)[side]
            o = (ou_hbm, os_hbm)[side]
            iv = (iu_v, is_v)[side]
            idx = iv.at[pl.ds(j * gch, gch)]
            return t.at[idx], o.at[pl.ds(base + j * gch, gch)]

        for k_ in range(nbuf):
            src, _ = chunk(k_)
            pltpu.async_copy(src, bufs[k_], sem_g[k_])
        for k_ in range(nch):
            b = k_ % nbuf
            src, dst = chunk(k_)
            pltpu.make_async_copy(src, bufs[b], sem_g[b]).wait()
            pltpu.async_copy(bufs[b], dst, sem_w[b])
            if k_ + nbuf < nch:
                pltpu.make_async_copy(bufs[b], dst, sem_w[b]).wait()
                nsrc, _ = chunk(k_ + nbuf)
                pltpu.async_copy(nsrc, bufs[b], sem_g[b])
        for k_ in range(nch - nbuf, nch):
            b = k_ % nbuf
            _, dst = chunk(k_)
            pltpu.make_async_copy(bufs[b], dst, sem_w[b]).wait()

    return k(tab_u, idx_u, tab_s, idx_s)


_LN_EPS = 1e-5


def _dense_body(acc_ref, deg_ref, feats_ref, w_ref, b_ref, g_ref, beta_ref,
                out_ref):
    agg = acc_ref[0] + acc_ref[1] + feats_ref[...]
    deg = deg_ref[0] + deg_ref[1] + 1.0
    h = agg / deg
    h = jnp.dot(h, w_ref[...], preferred_element_type=jnp.float32) + b_ref[...]
    mu = jnp.mean(h, axis=-1, keepdims=True)
    var = jnp.mean((h - mu) ** 2, axis=-1, keepdims=True)
    h = (h - mu) * lax.rsqrt(var + _LN_EPS) * g_ref[...] + beta_ref[...]
    out_ref[...] = jnp.where(h > 0, h, jnp.exp(jnp.minimum(h, 0.0)) - 1.0)


def _dense_tc(acc, deg, feats, w, b, g, beta, interpret=False):
    r = 400
    return pl.pallas_call(
        _dense_body,
        grid=(_N // r,),
        in_specs=[
            pl.BlockSpec((_NC, r, _D), lambda i: (0, i, 0)),
            pl.BlockSpec((_NC, r, 1), lambda i: (0, i, 0)),
            pl.BlockSpec((r, _D), lambda i: (i, 0)),
            pl.BlockSpec((_D, _D), lambda i: (0, 0)),
            pl.BlockSpec((1, _D), lambda i: (0, 0)),
            pl.BlockSpec((1, _D), lambda i: (0, 0)),
            pl.BlockSpec((1, _D), lambda i: (0, 0)),
        ],
        out_specs=pl.BlockSpec((r, _D), lambda i: (i, 0)),
        out_shape=jax.ShapeDtypeStruct((_N, _D), jnp.float32),
        interpret=interpret,
    )(acc, deg[..., None], feats, w, b.reshape(1, _D), g.reshape(1, _D),
      beta.reshape(1, _D))


def _mlp_body(u_ref, s_ref, w1u_ref, w1s_ref, b1_ref, g1_ref, beta1_ref,
              w2_ref, b2_ref, g2_ref, beta2_ref, w3_ref, b3_ref, out_ref):
    h = (jnp.dot(u_ref[...], w1u_ref[...], preferred_element_type=jnp.float32)
         + jnp.dot(s_ref[...], w1s_ref[...], preferred_element_type=jnp.float32)
         + b1_ref[...])
    mu = jnp.mean(h, axis=-1, keepdims=True)
    var = jnp.mean((h - mu) ** 2, axis=-1, keepdims=True)
    h = (h - mu) * lax.rsqrt(var + _LN_EPS) * g1_ref[...] + beta1_ref[...]
    h = jnp.maximum(h, 0.0)
    h = jnp.dot(h, w2_ref[...], preferred_element_type=jnp.float32) + b2_ref[...]
    mu = jnp.mean(h, axis=-1, keepdims=True)
    var = jnp.mean((h - mu) ** 2, axis=-1, keepdims=True)
    h = (h - mu) * lax.rsqrt(var + _LN_EPS) * g2_ref[...] + beta2_ref[...]
    h = jnp.maximum(h, 0.0)
    z = jnp.dot(h, w3_ref[...], preferred_element_type=jnp.float32) + b3_ref[...]
    out_ref[...] = jax.nn.sigmoid(z)


def _mlp_tc(u, s, w1, b1, g1, beta1, w2, b2, g2, beta2, w3, b3,
            interpret=False):
    r = 2048
    hid = w2.shape[0]
    return pl.pallas_call(
        _mlp_body,
        grid=(_B // r,),
        in_specs=[
            pl.BlockSpec((r, _D), lambda i: (i, 0)),
            pl.BlockSpec((r, _D), lambda i: (i, 0)),
            pl.BlockSpec((_D, hid), lambda i: (0, 0)),
            pl.BlockSpec((_D, hid), lambda i: (0, 0)),
            pl.BlockSpec((1, hid), lambda i: (0, 0)),
            pl.BlockSpec((1, hid), lambda i: (0, 0)),
            pl.BlockSpec((1, hid), lambda i: (0, 0)),
            pl.BlockSpec((hid, hid), lambda i: (0, 0)),
            pl.BlockSpec((1, hid), lambda i: (0, 0)),
            pl.BlockSpec((1, hid), lambda i: (0, 0)),
            pl.BlockSpec((1, hid), lambda i: (0, 0)),
            pl.BlockSpec((hid, 1), lambda i: (0, 0)),
            pl.BlockSpec((1, 1), lambda i: (0, 0)),
        ],
        out_specs=pl.BlockSpec((r, 1), lambda i: (i, 0)),
        out_shape=jax.ShapeDtypeStruct((_B, 1), jnp.float32),
        interpret=interpret,
    )(u, s, w1[:_D], w1[_D:], b1.reshape(1, -1), g1.reshape(1, -1),
      beta1.reshape(1, -1), w2, b2.reshape(1, -1), g2.reshape(1, -1),
      beta2.reshape(1, -1), w3, b3.reshape(1, 1))


def kernel(params, user_edges, serv_edges, userIdx, itemIdx):
    p = params
    e = {"user": (user_edges[0].reshape(_NROW, _CH),
                  user_edges[1].reshape(_NROW, _CH)),
         "serv": (serv_edges[0].reshape(_NROW, _CH),
                  serv_edges[1].reshape(_NROW, _CH))}
    feats = {"user": p["user_emb"], "serv": p["serv_emb"]}
    deg = {}

    def dense(side, l, acc):
        return _dense_tc(acc, deg[side], feats[side], p[f"{side}_W{l}"],
                         p[f"{side}_b{l}"], p[f"{side}_g{l}"],
                         p[f"{side}_beta{l}"])

    acc_u, d = _seg_sum_sc(feats["user"], *e["user"], True)
    deg["user"] = d.reshape(_NC, _NP)
    acc_s, d = _seg_sum_sc(feats["serv"], *e["serv"], True)
    deg["serv"] = d.reshape(_NC, _NP)
    feats["user"] = dense("user", 0, acc_u)
    (acc_u,) = _seg_sum_sc(feats["user"], *e["user"], False)
    feats["serv"] = dense("serv", 0, acc_s)
    (acc_s,) = _seg_sum_sc(feats["serv"], *e["serv"], False)
    feats["user"] = dense("user", 1, acc_u)
    feats["serv"] = dense("serv", 1, acc_s)
    u, s = _gather2_sc(feats["user"], userIdx, feats["serv"], itemIdx)
    est = _mlp_tc(u, s, p["W1"], p["b1"], p["g1"], p["beta1"], p["W2"],
                  p["b2"], p["g2"], p["beta2"], p["W3"], p["b3"])
    return est.reshape(_B)


# speedup vs baseline: 7.7457x; 1.0105x over previous
import functools

import jax
import jax.numpy as jnp
from jax import lax
from jax.experimental import pallas as pl
from jax.experimental.pallas import tpu as pltpu
from jax.experimental.pallas import tpu_sc as plsc

_N = 10000
_D = 128
_E = 320000
_B = 16384
_NC = 2
_NS = 16
_NW = _NC * _NS
_CH = 125
_NCH = 80
_NROW = _E // _CH
_NP = 10240
_RPT = _NP // _NS
_ZB = 32
_NB = 2
_IB = 40


def _seg_sum_sc(feats, src2d, dst2d, with_deg):
    mesh = plsc.VectorSubcoreMesh(core_axis_name="c", subcore_axis_name="s")

    out_type = [jax.ShapeDtypeStruct((_NC, _NP, _D), jnp.float32)]
    if with_deg:
        out_type.append(jax.ShapeDtypeStruct((_NC * _NP,), jnp.float32))

    scratch = [
        pltpu.VMEM((_IB, _CH), jnp.int32),
        pltpu.VMEM((_IB, _CH), jnp.int32),
        pltpu.VMEM((_CH, _D), jnp.float32),
        pltpu.VMEM((_CH, _D), jnp.float32),
        pltpu.VMEM((128,), jnp.float32),
        pltpu.VMEM((_ZB, _D), jnp.float32),
        pltpu.VMEM((_RPT,), jnp.float32),
        pltpu.VMEM_SHARED((_NP, _D), jnp.float32),
        pltpu.VMEM_SHARED((_NP,), jnp.float32),
        pltpu.SemaphoreType.DMA,
        pltpu.SemaphoreType.DMA,
        pltpu.SemaphoreType.DMA,
        pltpu.SemaphoreType.DMA,
        pltpu.SemaphoreType.DMA,
        pltpu.SemaphoreType.DMA,
    ]

    @functools.partial(pl.kernel, out_type=tuple(out_type), mesh=mesh,
                       scratch_types=scratch)
    def k(feats_hbm, src_hbm, dst_hbm, acc_hbm, *rest):
        if with_deg:
            deg_hbm = rest[0]
            rest = rest[1:]
        (srcs_v, dsts_v, rows_a, rows_b, ones_v, zrow_v, zdeg_v,
         acc_sh, deg_sh, sem_m, sem_ga, sem_gb, sem_sa, sem_sb,
         sem_d) = rest
        cid = lax.axis_index("c")
        sid = lax.axis_index("s")
        wid = cid * _NS + sid
        bufs = ((rows_a, sem_ga, sem_sa), (rows_b, sem_gb, sem_sb))

        pltpu.async_copy(src_hbm.at[pl.ds(wid * _NCH, _IB)], srcs_v, sem_d)
        pltpu.async_copy(dst_hbm.at[pl.ds(wid * _NCH, _IB)], dsts_v, sem_d)

        z16 = jnp.zeros((16,), jnp.float32)

        @pl.loop(0, _ZB)
        def _(r):
            @pl.loop(0, _D, step=16)
            def _(c):
                zrow_v[r, pl.ds(c, 16)] = z16

        if with_deg:
            o16 = jnp.ones((16,), jnp.float32)

            @pl.loop(0, _RPT, step=16)
            def _(i):
                zdeg_v[pl.ds(i, 16)] = z16

            @pl.loop(0, 128, step=16)
            def _(i):
                ones_v[pl.ds(i, 16)] = o16

        for i in range(_RPT // _ZB):
            pltpu.async_copy(
                zrow_v, acc_sh.at[pl.ds(sid * _RPT + i * _ZB, _ZB)], sem_m)
        if with_deg:
            pltpu.async_copy(zdeg_v, deg_sh.at[pl.ds(sid * _RPT, _RPT)],
                             sem_m)
        pltpu.make_async_copy(src_hbm.at[pl.ds(wid * _NCH, _IB)], srcs_v,
                              sem_d).wait()
        pltpu.make_async_copy(dst_hbm.at[pl.ds(wid * _NCH, _IB)], dsts_v,
                              sem_d).wait()
        for b, (buf, sem_g, _) in enumerate(bufs):
            pltpu.async_copy(feats_hbm.at[srcs_v.at[b]], buf, sem_g)

        for i in range(_RPT // _ZB):
            pltpu.make_async_copy(
                zrow_v, acc_sh.at[pl.ds(sid * _RPT + i * _ZB, _ZB)],
                sem_m).wait()
        if with_deg:
            pltpu.make_async_copy(zdeg_v, deg_sh.at[pl.ds(sid * _RPT, _RPT)],
                                  sem_m).wait()

        plsc.subcore_barrier()

        @pl.loop(0, _NCH, step=_IB)
        def _(t0):
            @pl.when(t0 > 0)
            def _():
                row0 = wid * _NCH + t0
                pltpu.sync_copy(src_hbm.at[pl.ds(row0, _IB)], srcs_v)
                pltpu.sync_copy(dst_hbm.at[pl.ds(row0, _IB)], dsts_v)
                for b, (buf, sem_g, _) in enumerate(bufs):
                    pltpu.async_copy(feats_hbm.at[srcs_v.at[b]], buf, sem_g)

            @pl.loop(0, _IB, step=_NB)
            def _(t):
                for b, (buf, sem_g, sem_s) in enumerate(bufs):
                    j = t + b
                    pltpu.make_async_copy(feats_hbm.at[srcs_v.at[j]], buf,
                                          sem_g).wait()
                    pltpu.async_copy(buf, acc_sh.at[dsts_v.at[j]], sem_s,
                                     add=True)
                    if with_deg:
                        pltpu.async_copy(ones_v.at[pl.ds(0, _CH)],
                                         deg_sh.at[dsts_v.at[j]], sem_d,
                                         add=True)
                for b, (buf, sem_g, sem_s) in enumerate(bufs):
                    j = t + b
                    pltpu.make_async_copy(buf, acc_sh.at[dsts_v.at[j]],
                                          sem_s).wait()

                    @pl.when(t + _NB < _IB)
                    def _(j=j, buf=buf, sem_g=sem_g):
                        pltpu.async_copy(feats_hbm.at[srcs_v.at[j + _NB]],
                                         buf, sem_g)

            if with_deg:
                @pl.loop(0, _IB)
                def _(j):
                    pltpu.make_async_copy(ones_v.at[pl.ds(0, _CH)],
                                          deg_sh.at[dsts_v.at[j]],
                                          sem_d).wait()

        plsc.subcore_barrier()

        pltpu.sync_copy(acc_sh.at[pl.ds(sid * _RPT, _RPT)],
                        acc_hbm.at[cid, pl.ds(sid * _RPT, _RPT)])
        if with_deg:
            pltpu.sync_copy(deg_sh.at[pl.ds(sid * _RPT, _RPT)],
                            deg_hbm.at[pl.ds(cid * _NP + sid * _RPT, _RPT)])

    return k(feats, src2d, dst2d)


def _gather2_sc(tab_u, idx_u, tab_s, idx_s):
    mesh = plsc.VectorSubcoreMesh(core_axis_name="c", subcore_axis_name="s")
    ipw = _B // _NW
    gch = 128
    nbuf = 6
    nch = 2 * (ipw // gch)

    scratch = ([pltpu.VMEM((ipw,), jnp.int32)] * 2 +
               [pltpu.VMEM((gch, _D), jnp.float32)] * nbuf +
               [pltpu.SemaphoreType.DMA] * (1 + 2 * nbuf))

    @functools.partial(
        pl.kernel,
        out_type=(jax.ShapeDtypeStruct((_B, _D), jnp.float32),
                  jax.ShapeDtypeStruct((_B, _D), jnp.float32)),
        mesh=mesh,
        scratch_types=scratch,
    )
    def k(tu_hbm, iu_hbm, ts_hbm, is_hbm, ou_hbm, os_hbm, *rest):
        iu_v, is_v = rest[0], rest[1]
        bufs = rest[2:2 + nbuf]
        sem_m = rest[2 + nbuf]
        sem_g = rest[3 + nbuf:3 + 2 * nbuf]
        sem_w = rest[3 + 2 * nbuf:3 + 3 * nbuf]
        cid = lax.axis_index("c")
        sid = lax.axis_index("s")
        wid = cid * _NS + sid
        base = wid * ipw

        pltpu.async_copy(iu_hbm.at[pl.ds(base, ipw)], iu_v, sem_m)
        pltpu.async_copy(is_hbm.at[pl.ds(base, ipw)], is_v, sem_m)
        pltpu.make_async_copy(iu_hbm.at[pl.ds(base, ipw)], iu_v, sem_m).wait()
        pltpu.make_async_copy(is_hbm.at[pl.ds(base, ipw)], is_v, sem_m).wait()

        def chunk(k_):
            side = k_ // (nch // 2)
            j = k_ % (nch // 2)
            t = (tu_hbm, ts_hbm)[side]
            o = (ou_hbm, os_hbm)[side]
            iv = (iu_v, is_v)[side]
            idx = iv.at[pl.ds(j * gch, gch)]
            return t.at[idx], o.at[pl.ds(base + j * gch, gch)]

        for k_ in range(nbuf):
            src, _ = chunk(k_)
            pltpu.async_copy(src, bufs[k_], sem_g[k_])
        for k_ in range(nch):
            b = k_ % nbuf
            src, dst = chunk(k_)
            pltpu.make_async_copy(src, bufs[b], sem_g[b]).wait()
            pltpu.async_copy(bufs[b], dst, sem_w[b])
            if k_ + nbuf < nch:
                pltpu.make_async_copy(bufs[b], dst, sem_w[b]).wait()
                nsrc, _ = chunk(k_ + nbuf)
                pltpu.async_copy(nsrc, bufs[b], sem_g[b])
        for k_ in range(nch - nbuf, nch):
            b = k_ % nbuf
            _, dst = chunk(k_)
            pltpu.make_async_copy(bufs[b], dst, sem_w[b]).wait()

    return k(tab_u, idx_u, tab_s, idx_s)


_LN_EPS = 1e-5


def _dense_body(acc_ref, deg_ref, feats_ref, w_ref, b_ref, g_ref, beta_ref,
                out_ref):
    agg = acc_ref[0] + acc_ref[1] + feats_ref[...]
    deg = deg_ref[0] + deg_ref[1] + 1.0
    h = agg / deg
    h = jnp.dot(h, w_ref[...], preferred_element_type=jnp.float32) + b_ref[...]
    mu = jnp.mean(h, axis=-1, keepdims=True)
    var = jnp.mean((h - mu) ** 2, axis=-1, keepdims=True)
    h = (h - mu) * lax.rsqrt(var + _LN_EPS) * g_ref[...] + beta_ref[...]
    out_ref[...] = jnp.where(h > 0, h, jnp.exp(jnp.minimum(h, 0.0)) - 1.0)


def _dense_tc(acc, deg, feats, w, b, g, beta, interpret=False):
    r = 2000
    return pl.pallas_call(
        _dense_body,
        grid=(_N // r,),
        in_specs=[
            pl.BlockSpec((_NC, r, _D), lambda i: (0, i, 0)),
            pl.BlockSpec((_NC, r, 1), lambda i: (0, i, 0)),
            pl.BlockSpec((r, _D), lambda i: (i, 0)),
            pl.BlockSpec((_D, _D), lambda i: (0, 0)),
            pl.BlockSpec((1, _D), lambda i: (0, 0)),
            pl.BlockSpec((1, _D), lambda i: (0, 0)),
            pl.BlockSpec((1, _D), lambda i: (0, 0)),
        ],
        out_specs=pl.BlockSpec((r, _D), lambda i: (i, 0)),
        out_shape=jax.ShapeDtypeStruct((_N, _D), jnp.float32),
        interpret=interpret,
    )(acc, deg[..., None], feats, w, b.reshape(1, _D), g.reshape(1, _D),
      beta.reshape(1, _D))


def _mlp_body(u_ref, s_ref, w1u_ref, w1s_ref, b1_ref, g1_ref, beta1_ref,
              w2_ref, b2_ref, g2_ref, beta2_ref, w3_ref, b3_ref, out_ref):
    h = (jnp.dot(u_ref[...], w1u_ref[...], preferred_element_type=jnp.float32)
         + jnp.dot(s_ref[...], w1s_ref[...], preferred_element_type=jnp.float32)
         + b1_ref[...])
    mu = jnp.mean(h, axis=-1, keepdims=True)
    var = jnp.mean((h - mu) ** 2, axis=-1, keepdims=True)
    h = (h - mu) * lax.rsqrt(var + _LN_EPS) * g1_ref[...] + beta1_ref[...]
    h = jnp.maximum(h, 0.0)
    h = jnp.dot(h, w2_ref[...], preferred_element_type=jnp.float32) + b2_ref[...]
    mu = jnp.mean(h, axis=-1, keepdims=True)
    var = jnp.mean((h - mu) ** 2, axis=-1, keepdims=True)
    h = (h - mu) * lax.rsqrt(var + _LN_EPS) * g2_ref[...] + beta2_ref[...]
    h = jnp.maximum(h, 0.0)
    z = jnp.dot(h, w3_ref[...], preferred_element_type=jnp.float32) + b3_ref[...]
    out_ref[...] = jax.nn.sigmoid(z)


def _mlp_tc(u, s, w1, b1, g1, beta1, w2, b2, g2, beta2, w3, b3,
            interpret=False):
    r = 2048
    hid = w2.shape[0]
    return pl.pallas_call(
        _mlp_body,
        grid=(_B // r,),
        in_specs=[
            pl.BlockSpec((r, _D), lambda i: (i, 0)),
            pl.BlockSpec((r, _D), lambda i: (i, 0)),
            pl.BlockSpec((_D, hid), lambda i: (0, 0)),
            pl.BlockSpec((_D, hid), lambda i: (0, 0)),
            pl.BlockSpec((1, hid), lambda i: (0, 0)),
            pl.BlockSpec((1, hid), lambda i: (0, 0)),
            pl.BlockSpec((1, hid), lambda i: (0, 0)),
            pl.BlockSpec((hid, hid), lambda i: (0, 0)),
            pl.BlockSpec((1, hid), lambda i: (0, 0)),
            pl.BlockSpec((1, hid), lambda i: (0, 0)),
            pl.BlockSpec((1, hid), lambda i: (0, 0)),
            pl.BlockSpec((hid, 1), lambda i: (0, 0)),
            pl.BlockSpec((1, 1), lambda i: (0, 0)),
        ],
        out_specs=pl.BlockSpec((r, 1), lambda i: (i, 0)),
        out_shape=jax.ShapeDtypeStruct((_B, 1), jnp.float32),
        interpret=interpret,
    )(u, s, w1[:_D], w1[_D:], b1.reshape(1, -1), g1.reshape(1, -1),
      beta1.reshape(1, -1), w2, b2.reshape(1, -1), g2.reshape(1, -1),
      beta2.reshape(1, -1), w3, b3.reshape(1, 1))


def kernel(params, user_edges, serv_edges, userIdx, itemIdx):
    p = params
    e = {"user": (user_edges[0].reshape(_NROW, _CH),
                  user_edges[1].reshape(_NROW, _CH)),
         "serv": (serv_edges[0].reshape(_NROW, _CH),
                  serv_edges[1].reshape(_NROW, _CH))}
    feats = {"user": p["user_emb"], "serv": p["serv_emb"]}
    deg = {}

    def dense(side, l, acc):
        return _dense_tc(acc, deg[side], feats[side], p[f"{side}_W{l}"],
                         p[f"{side}_b{l}"], p[f"{side}_g{l}"],
                         p[f"{side}_beta{l}"])

    acc_u, d = _seg_sum_sc(feats["user"], *e["user"], True)
    deg["user"] = d.reshape(_NC, _NP)
    acc_s, d = _seg_sum_sc(feats["serv"], *e["serv"], True)
    deg["serv"] = d.reshape(_NC, _NP)
    feats["user"] = dense("user", 0, acc_u)
    (acc_u,) = _seg_sum_sc(feats["user"], *e["user"], False)
    feats["serv"] = dense("serv", 0, acc_s)
    (acc_s,) = _seg_sum_sc(feats["serv"], *e["serv"], False)
    feats["user"] = dense("user", 1, acc_u)
    feats["serv"] = dense("serv", 1, acc_s)
    u, s = _gather2_sc(feats["user"], userIdx, feats["serv"], itemIdx)
    est = _mlp_tc(u, s, p["W1"], p["b1"], p["g1"], p["beta1"], p["W2"],
                  p["b2"], p["g2"], p["beta2"], p["W3"], p["b3"])
    return est.reshape(_B)
